# Initial kernel scaffold; baseline (speedup 1.0000x reference)
import jax
import jax.numpy as jnp
from jax.experimental import pallas as pl


def kernel(features, edge_index, edge_CSL, W1, att_src1, att_dst1, W2, Wd1, bd1, Wd2, bd2):
    raise NotImplementedError("write your pallas kernel here")


# speedup vs baseline: 7.5702x; 7.5702x over previous
import jax
import jax.numpy as jnp
from jax import lax
from jax.experimental import pallas as pl
from jax.experimental.pallas import tpu as pltpu
from jax.experimental.pallas import tpu_sc as plsc

N = 10000
E = 160000
IN_DIM, HID, OUT = 256, 512, 64
HALF = IN_DIM // 2
NS = 16
NP = 10240
RPT = NP // NS
CW = 128
CPT = NP // CW
WPT = NP // 2 // CW
F32 = jnp.float32
I32 = jnp.int32


def _zvec():
    return jnp.zeros((16,), F32)


def _tc1_body(x_ref, w1_ref, att_ref, out_ref):
    wmat = jnp.dot(w1_ref[...], att_ref[...], preferred_element_type=F32)
    out_ref[...] = jnp.dot(x_ref[...], wmat, preferred_element_type=F32)


def _tc1(features, W1, att2p):
    return pl.pallas_call(
        _tc1_body,
        grid=(25,),
        in_specs=[
            pl.BlockSpec((400, IN_DIM), lambda i: (i, 0)),
            pl.BlockSpec((IN_DIM, HID), lambda i: (0, 0)),
            pl.BlockSpec((HID, 128), lambda i: (0, 0)),
        ],
        out_specs=pl.BlockSpec((400, 128), lambda i: (i, 0)),
        out_shape=jax.ShapeDtypeStruct((N, 128), F32),
    )(features, W1, att2p)


def _gat_sc_body(fcat, asrc_h, adst_h, src_h, dst_h, out,
                 src_t, dst_t, w_t, rbuf, av_t, bv_t, zden_t,
                 den_s, agg_s):
    c = lax.axis_index("c")
    s = lax.axis_index("s")
    row0 = s * RPT

    pltpu.sync_copy(src_h.at[s], src_t)
    pltpu.sync_copy(dst_h.at[s], dst_t)

    def zd(i, _):
        zden_t[pl.ds(i * 16, 16)] = _zvec()
        return 0
    lax.fori_loop(0, RPT // 16, zd, 0)
    pltpu.sync_copy(zden_t, den_s.at[pl.ds(row0, RPT)])

    def zr(i, _):
        for v in range(8):
            rbuf[i, pl.ds(v * 16, 16)] = _zvec()
        return 0
    lax.fori_loop(0, CW, zr, 0)
    for k in range(RPT // CW):
        pltpu.sync_copy(rbuf, agg_s.at[pl.ds(row0 + k * CW, CW)])

    plsc.subcore_barrier()

    def p1(r, _):
        pltpu.sync_copy(asrc_h.at[src_t.at[r]], av_t)
        pltpu.sync_copy(adst_h.at[dst_t.at[r]], bv_t)
        for k in range(CW // 16):
            sl = pl.ds(k * 16, 16)
            e = av_t[sl] + bv_t[sl]
            e = jnp.where(e > 0.0, e, e * jnp.float32(0.2))
            w_t[r, sl] = jnp.exp(e)
        pltpu.sync_copy(w_t.at[r], den_s.at[dst_t.at[r]], add=True)
        return 0
    lax.fori_loop(0, CPT, p1, 0)

    plsc.subcore_barrier()

    coff = c * NP

    def p2(r, _):
        pltpu.sync_copy(den_s.at[dst_t.at[r]], av_t)
        for k in range(CW // 16):
            sl = pl.ds(k * 16, 16)
            w_t[r, sl] = w_t[r, sl] / (av_t[sl] + jnp.float32(1e-16))
            src_t[r, sl] = src_t[r, sl] + coff
        return 0
    lax.fori_loop(0, CPT, p2, 0)

    def p3(g, _):
        pltpu.sync_copy(fcat.at[src_t.at[g]], rbuf)

        def scale(k, _):
            wv = w_t[g, pl.ds(k * 16, 16)]
            for j in range(16):
                wj = wv[j]
                e = k * 16 + j
                for v in range(8):
                    sl = pl.ds(v * 16, 16)
                    rbuf[e, sl] = rbuf[e, sl] * wj
            return 0
        lax.fori_loop(0, CW // 16, scale, 0)
        pltpu.sync_copy(rbuf, agg_s.at[dst_t.at[g]], add=True)
        return 0
    lax.fori_loop(0, CPT, p3, 0)

    plsc.subcore_barrier()

    pltpu.sync_copy(agg_s.at[pl.ds(row0, RPT)], out.at[c, pl.ds(row0, RPT)])


def _gat_sc(fcat, asrc, adst, src3d, dst3d):
    mesh = plsc.VectorSubcoreMesh(core_axis_name="c", subcore_axis_name="s")
    f = pl.kernel(
        _gat_sc_body,
        out_type=jax.ShapeDtypeStruct((2, NP, HALF), F32),
        mesh=mesh,
        compiler_params=pltpu.CompilerParams(needs_layout_passes=False),
        scratch_types=dict(
            src_t=pltpu.VMEM((CPT, CW), I32),
            dst_t=pltpu.VMEM((CPT, CW), I32),
            w_t=pltpu.VMEM((CPT, CW), F32),
            rbuf=pltpu.VMEM((CW, HALF), F32),
            av_t=pltpu.VMEM((CW,), F32),
            bv_t=pltpu.VMEM((CW,), F32),
            zden_t=pltpu.VMEM((RPT,), F32),
            den_s=pltpu.VMEM_SHARED((NP,), F32),
            agg_s=pltpu.VMEM_SHARED((NP, HALF), F32),
        ),
    )
    return f(fcat, asrc, adst, src3d, dst3d)


def _elu(x):
    return jnp.where(x > 0.0, x, jnp.exp(x) - 1.0)


def _tc2_body(apl_ref, aph_ref, anl_ref, anh_ref, w1_ref, w2_ref,
              wd1_ref, bd1_ref, wd2_ref, bd2_ref,
              h2_ref, h2n_ref, rec_ref):
    w1l = w1_ref[0:HALF, :]
    w1h = w1_ref[HALF:IN_DIM, :]
    h1 = jnp.dot(apl_ref[0], w1l, preferred_element_type=F32)
    h1 = h1 + jnp.dot(aph_ref[0], w1h, preferred_element_type=F32)
    h2 = jnp.dot(_elu(h1), w2_ref[...], preferred_element_type=F32)
    h1n = jnp.dot(anl_ref[0], w1l, preferred_element_type=F32)
    h1n = h1n + jnp.dot(anh_ref[0], w1h, preferred_element_type=F32)
    h2n = jnp.dot(_elu(h1n), w2_ref[...], preferred_element_type=F32)
    r1 = _elu(jnp.dot(h2, wd1_ref[...], preferred_element_type=F32)
              + bd1_ref[...])
    rec = jnp.dot(r1, wd2_ref[...], preferred_element_type=F32) + bd2_ref[...]
    zpad = jnp.zeros((h2.shape[0], 128 - OUT), F32)
    h2_ref[...] = jnp.concatenate([h2, zpad], axis=1)
    h2n_ref[...] = jnp.concatenate([h2n, zpad], axis=1)
    rec_ref[...] = rec


def _tc2(aggP, aggN, W1, W2, Wd1, bd1r, Wd2, bd2r):
    blk = 512
    lo = lambda i: (0, i, 0)
    hi = lambda i: (1, i, 0)

    def full(shape):
        return pl.BlockSpec(shape, lambda i: tuple(0 for _ in shape))

    return pl.pallas_call(
        _tc2_body,
        grid=(NP // blk,),
        in_specs=[
            pl.BlockSpec((1, blk, HALF), lo),
            pl.BlockSpec((1, blk, HALF), hi),
            pl.BlockSpec((1, blk, HALF), lo),
            pl.BlockSpec((1, blk, HALF), hi),
            full((IN_DIM, HID)),
            full((HID, OUT)),
            full((OUT, HID)),
            full((1, HID)),
            full((HID, IN_DIM)),
            full((1, IN_DIM)),
        ],
        out_specs=[
            pl.BlockSpec((blk, 128), lambda i: (i, 0)),
            pl.BlockSpec((blk, 128), lambda i: (i, 0)),
            pl.BlockSpec((blk, IN_DIM), lambda i: (i, 0)),
        ],
        out_shape=[
            jax.ShapeDtypeStruct((NP, 128), F32),
            jax.ShapeDtypeStruct((NP, 128), F32),
            jax.ShapeDtypeStruct((NP, IN_DIM), F32),
        ],
    )(aggP, aggP, aggN, aggN, W1, W2, Wd1, bd1r, Wd2, bd2r)


def _csl_sc_body(h2pad, src_h, dst_h, acc_out, cnt_out,
                 src_t, dst_t, rbuf, ones_t, zden_t, acc_s, cnt_s):
    c = lax.axis_index("c")
    s = lax.axis_index("s")
    row0 = s * RPT
    w = c * NS + s

    pltpu.sync_copy(src_h.at[w], src_t)
    pltpu.sync_copy(dst_h.at[w], dst_t)

    def zd(i, _):
        zden_t[pl.ds(i * 16, 16)] = _zvec()
        return 0
    lax.fori_loop(0, RPT // 16, zd, 0)
    pltpu.sync_copy(zden_t, cnt_s.at[pl.ds(row0, RPT)])

    def zr(i, _):
        for v in range(128 // 16):
            rbuf[i, pl.ds(v * 16, 16)] = _zvec()
        return 0
    lax.fori_loop(0, CW, zr, 0)
    for k in range(RPT // CW):
        pltpu.sync_copy(rbuf, acc_s.at[pl.ds(row0 + k * CW, CW)])

    for k in range(CW // 16):
        ones_t[pl.ds(k * 16, 16)] = jnp.ones((16,), F32)

    plsc.subcore_barrier()

    def p1(g, _):
        pltpu.sync_copy(h2pad.at[dst_t.at[g]], rbuf)
        pltpu.sync_copy(rbuf, acc_s.at[src_t.at[g]], add=True)
        pltpu.sync_copy(ones_t, cnt_s.at[src_t.at[g]], add=True)
        return 0
    lax.fori_loop(0, WPT, p1, 0)

    plsc.subcore_barrier()

    pltpu.sync_copy(acc_s.at[pl.ds(row0, RPT)], acc_out.at[c, pl.ds(row0, RPT)])
    pltpu.sync_copy(cnt_s.at[pl.ds(row0, RPT)], cnt_out.at[c, pl.ds(row0, RPT)])


def _csl_sc(h2pad, src3d, dst3d):
    mesh = plsc.VectorSubcoreMesh(core_axis_name="c", subcore_axis_name="s")
    f = pl.kernel(
        _csl_sc_body,
        out_type=[
            jax.ShapeDtypeStruct((2, NP, 128), F32),
            jax.ShapeDtypeStruct((2, NP), F32),
        ],
        mesh=mesh,
        compiler_params=pltpu.CompilerParams(needs_layout_passes=False),
        scratch_types=dict(
            src_t=pltpu.VMEM((WPT, CW), I32),
            dst_t=pltpu.VMEM((WPT, CW), I32),
            rbuf=pltpu.VMEM((CW, 128), F32),
            ones_t=pltpu.VMEM((CW,), F32),
            zden_t=pltpu.VMEM((RPT,), F32),
            acc_s=pltpu.VMEM_SHARED((NP, 128), F32),
            cnt_s=pltpu.VMEM_SHARED((NP,), F32),
        ),
    )
    return f(h2pad, src3d, dst3d)


def _tc3_body(a_lo, a_hi, c_lo, c_hi, out_ref):
    cnt = (c_lo[0] + c_hi[0]).reshape(-1)
    inv = 1.0 / jnp.maximum(cnt, 1.0)
    out_ref[...] = (a_lo[0] + a_hi[0]) * inv.reshape(-1, 1)


def _tc3(acc2, cnt2):
    blk = 1024
    lo = lambda i: (0, i, 0)
    hi = lambda i: (1, i, 0)
    cnt3 = cnt2.reshape(2, NP // CW, CW)
    return pl.pallas_call(
        _tc3_body,
        grid=(NP // blk,),
        in_specs=[
            pl.BlockSpec((1, blk, 128), lo),
            pl.BlockSpec((1, blk, 128), hi),
            pl.BlockSpec((1, blk // CW, CW), lo),
            pl.BlockSpec((1, blk // CW, CW), hi),
        ],
        out_specs=pl.BlockSpec((blk, 128), lambda i: (i, 0)),
        out_shape=jax.ShapeDtypeStruct((NP, 128), F32),
    )(acc2, acc2, cnt3, cnt3)


def _pad_edges(idx):
    blocks = idx.reshape(NS, E // NS)
    blocks = jnp.pad(blocks, ((0, 0), (0, NP - E // NS)),
                     constant_values=NP - 1)
    return blocks.reshape(NS, CPT, CW)


def kernel(features, edge_index, edge_CSL, W1, att_src1, att_dst1, W2,
           Wd1, bd1, Wd2, bd2):
    att2p = jnp.zeros((HID, 128), F32)
    att2p = att2p.at[:, 0].set(att_src1).at[:, 1].set(att_dst1)
    a2 = _tc1(features, W1, att2p)
    asrc = jnp.pad(a2[:, 0], (0, NP - N))
    adst = jnp.pad(a2[:, 1], (0, NP - N))

    fcat = jnp.zeros((2 * NP, HALF), F32)
    fcat = fcat.at[0:N].set(features[:, :HALF])
    fcat = fcat.at[NP:NP + N].set(features[:, HALF:])

    srcP = _pad_edges(edge_index[0])
    dstP = _pad_edges(edge_index[1])
    srcN = _pad_edges(edge_CSL[0])
    dstN = _pad_edges(edge_CSL[1])

    aggP = _gat_sc(fcat, asrc, adst, srcP, dstP)
    aggN = _gat_sc(fcat, asrc, adst, srcN, dstN)

    h2p, h2np, recp = _tc2(aggP, aggN, W1, W2, Wd1, bd1.reshape(1, HID),
                           Wd2, bd2.reshape(1, IN_DIM))

    acc2, cnt2 = _csl_sc(h2p,
                         srcP.reshape(2 * NS, WPT, CW),
                         dstP.reshape(2 * NS, WPT, CW))
    hp = _tc3(acc2, cnt2)

    return h2p[:N, :OUT], hp[:N, :OUT], h2np[:N, :OUT], recp[:N]


# speedup vs baseline: 7.7384x; 1.0222x over previous
import jax
import jax.numpy as jnp
from jax import lax
from jax.experimental import pallas as pl
from jax.experimental.pallas import tpu as pltpu
from jax.experimental.pallas import tpu_sc as plsc

N = 10000
E = 160000
IN_DIM, HID, OUT = 256, 512, 64
HALF = IN_DIM // 2
NS = 16
NP = 10240
RPT = NP // NS
CW = 128
CPT = NP // CW
WPT = NP // 2 // CW
F32 = jnp.float32
I32 = jnp.int32


def _zvec():
    return jnp.zeros((16,), F32)


def _tc1_body(x_ref, w1_ref, att_ref, out_ref):
    wmat = jnp.dot(w1_ref[...], att_ref[...], preferred_element_type=F32)
    out_ref[...] = jnp.dot(x_ref[...], wmat, preferred_element_type=F32)


def _tc1(features, W1, att2p):
    return pl.pallas_call(
        _tc1_body,
        grid=(25,),
        in_specs=[
            pl.BlockSpec((400, IN_DIM), lambda i: (i, 0)),
            pl.BlockSpec((IN_DIM, HID), lambda i: (0, 0)),
            pl.BlockSpec((HID, 128), lambda i: (0, 0)),
        ],
        out_specs=pl.BlockSpec((400, 128), lambda i: (i, 0)),
        out_shape=jax.ShapeDtypeStruct((N, 128), F32),
    )(features, W1, att2p)


def _gat_sc_body(fcat, asrc_h, adst_h, src_h, dst_h, out, den_out,
                 src_l, dst2, w_l, rbuf, den_s, agg_s):
    c = lax.axis_index("c")
    s = lax.axis_index("s")
    row0 = s * RPT
    EP = NP

    pltpu.sync_copy(src_h.at[pl.ds(s * EP, EP)], src_l)
    pltpu.sync_copy(dst_h.at[s], dst2)

    def zd(i, _):
        w_l[pl.ds(i * 16, 16)] = _zvec()
        return 0
    lax.fori_loop(0, RPT // 16, zd, 0)
    pltpu.sync_copy(w_l.at[pl.ds(0, RPT)], den_s.at[pl.ds(row0, RPT)])

    def zr(i, _):
        for v in range(8):
            rbuf[i, pl.ds(v * 16, 16)] = _zvec()
        return 0
    lax.fori_loop(0, CW, zr, 0)
    for k in range(RPT // CW):
        pltpu.sync_copy(rbuf, agg_s.at[pl.ds(row0 + k * CW, CW)])

    plsc.subcore_barrier()

    def p1(r, _):
        sl_e = pl.ds(r * CW, CW)
        pltpu.sync_copy(asrc_h.at[src_l.at[sl_e]], w_l.at[sl_e])
        pltpu.sync_copy(adst_h.at[dst2.at[r]], rbuf.at[0])
        for k in range(CW // 16):
            sl = pl.ds(r * CW + k * 16, 16)
            e = w_l[sl] + rbuf[0, pl.ds(k * 16, 16)]
            e = jnp.where(e > 0.0, e, e * jnp.float32(0.2))
            w_l[sl] = jnp.exp(e)
        pltpu.sync_copy(w_l.at[sl_e], den_s.at[dst2.at[r]], add=True)
        return 0
    lax.fori_loop(0, CPT, p1, 0)

    coff = c * NP

    def padj(i, _):
        sl = pl.ds(i * 16, 16)
        src_l[sl] = src_l[sl] + coff
        return 0
    lax.fori_loop(0, EP // 16, padj, 0)

    plsc.subcore_barrier()

    def p3(g, _):
        pltpu.sync_copy(fcat.at[src_l.at[pl.ds(g * CW, CW)]], rbuf)

        def scale(k, _):
            wv = w_l[pl.ds(g * CW + k * 16, 16)]
            for j in range(16):
                wj = wv[j]
                e = k * 16 + j
                for v in range(8):
                    sl = pl.ds(v * 16, 16)
                    rbuf[e, sl] = rbuf[e, sl] * wj
            return 0
        lax.fori_loop(0, CW // 16, scale, 0)
        pltpu.sync_copy(rbuf, agg_s.at[dst2.at[g]], add=True)
        return 0
    lax.fori_loop(0, CPT, p3, 0)

    plsc.subcore_barrier()

    pltpu.sync_copy(agg_s.at[pl.ds(row0, RPT)], out.at[c, pl.ds(row0, RPT)])
    pltpu.sync_copy(den_s.at[pl.ds(row0, RPT)], den_out.at[c, pl.ds(row0, RPT)])


def _gat_sc(fcat, asrc, adst, srcF, dst3):
    mesh = plsc.VectorSubcoreMesh(core_axis_name="c", subcore_axis_name="s")
    f = pl.kernel(
        _gat_sc_body,
        out_type=[
            jax.ShapeDtypeStruct((2, NP, HALF), F32),
            jax.ShapeDtypeStruct((2, NP), F32),
        ],
        mesh=mesh,
        compiler_params=pltpu.CompilerParams(needs_layout_passes=False),
        scratch_types=dict(
            src_l=pltpu.VMEM((NP,), I32),
            dst2=pltpu.VMEM((CPT, CW), I32),
            w_l=pltpu.VMEM((NP,), F32),
            rbuf=pltpu.VMEM((CW, HALF), F32),
            den_s=pltpu.VMEM_SHARED((NP,), F32),
            agg_s=pltpu.VMEM_SHARED((NP, HALF), F32),
        ),
    )
    return f(fcat, asrc, adst, srcF, dst3)


def _elu(x):
    return jnp.where(x > 0.0, x, jnp.exp(x) - 1.0)


def _tc2_body(apl_ref, aph_ref, anl_ref, anh_ref, dp_ref, dn_ref,
              w1_ref, w2_ref, wd1_ref, bd1_ref, wd2_ref, bd2_ref,
              h2_ref, h2n_ref, rec_ref):
    w1l = w1_ref[0:HALF, :]
    w1h = w1_ref[HALF:IN_DIM, :]
    ivp = (1.0 / (dp_ref[...].reshape(-1) + jnp.float32(1e-16))).reshape(-1, 1)
    ivn = (1.0 / (dn_ref[...].reshape(-1) + jnp.float32(1e-16))).reshape(-1, 1)
    h1 = jnp.dot(apl_ref[0] * ivp, w1l, preferred_element_type=F32)
    h1 = h1 + jnp.dot(aph_ref[0] * ivp, w1h, preferred_element_type=F32)
    h2 = jnp.dot(_elu(h1), w2_ref[...], preferred_element_type=F32)
    h1n = jnp.dot(anl_ref[0] * ivn, w1l, preferred_element_type=F32)
    h1n = h1n + jnp.dot(anh_ref[0] * ivn, w1h, preferred_element_type=F32)
    h2n = jnp.dot(_elu(h1n), w2_ref[...], preferred_element_type=F32)
    r1 = _elu(jnp.dot(h2, wd1_ref[...], preferred_element_type=F32)
              + bd1_ref[...])
    rec = jnp.dot(r1, wd2_ref[...], preferred_element_type=F32) + bd2_ref[...]
    zpad = jnp.zeros((h2.shape[0], 128 - OUT), F32)
    h2_ref[...] = jnp.concatenate([h2, zpad], axis=1)
    h2n_ref[...] = jnp.concatenate([h2n, zpad], axis=1)
    rec_ref[...] = rec


def _tc2(aggP, denP, aggN, denN, W1, W2, Wd1, bd1r, Wd2, bd2r):
    blk = 1024
    lo = lambda i: (0, i, 0)
    hi = lambda i: (1, i, 0)
    dP = denP[0].reshape(NP // CW, CW)
    dN = denN[0].reshape(NP // CW, CW)

    def full(shape):
        return pl.BlockSpec(shape, lambda i: tuple(0 for _ in shape))

    return pl.pallas_call(
        _tc2_body,
        grid=(NP // blk,),
        in_specs=[
            pl.BlockSpec((1, blk, HALF), lo),
            pl.BlockSpec((1, blk, HALF), hi),
            pl.BlockSpec((1, blk, HALF), lo),
            pl.BlockSpec((1, blk, HALF), hi),
            pl.BlockSpec((blk // CW, CW), lambda i: (i, 0)),
            pl.BlockSpec((blk // CW, CW), lambda i: (i, 0)),
            full((IN_DIM, HID)),
            full((HID, OUT)),
            full((OUT, HID)),
            full((1, HID)),
            full((HID, IN_DIM)),
            full((1, IN_DIM)),
        ],
        out_specs=[
            pl.BlockSpec((blk, 128), lambda i: (i, 0)),
            pl.BlockSpec((blk, 128), lambda i: (i, 0)),
            pl.BlockSpec((blk, IN_DIM), lambda i: (i, 0)),
        ],
        out_shape=[
            jax.ShapeDtypeStruct((NP, 128), F32),
            jax.ShapeDtypeStruct((NP, 128), F32),
            jax.ShapeDtypeStruct((NP, IN_DIM), F32),
        ],
    )(aggP, aggP, aggN, aggN, dP, dN, W1, W2, Wd1, bd1r, Wd2, bd2r)


def _csl_sc_body(h2pad, src_h, dst_h, acc_out, cnt_out,
                 src_t, dst_t, rbuf, ones_t, zden_t, acc_s, cnt_s):
    c = lax.axis_index("c")
    s = lax.axis_index("s")
    row0 = s * RPT
    w = c * NS + s

    pltpu.sync_copy(src_h.at[w], src_t)
    pltpu.sync_copy(dst_h.at[w], dst_t)

    def zd(i, _):
        zden_t[pl.ds(i * 16, 16)] = _zvec()
        return 0
    lax.fori_loop(0, RPT // 16, zd, 0)
    pltpu.sync_copy(zden_t, cnt_s.at[pl.ds(row0, RPT)])

    def zr(i, _):
        for v in range(128 // 16):
            rbuf[i, pl.ds(v * 16, 16)] = _zvec()
        return 0
    lax.fori_loop(0, CW, zr, 0)
    for k in range(RPT // CW):
        pltpu.sync_copy(rbuf, acc_s.at[pl.ds(row0 + k * CW, CW)])

    for k in range(CW // 16):
        ones_t[pl.ds(k * 16, 16)] = jnp.ones((16,), F32)

    plsc.subcore_barrier()

    def p1(g, _):
        pltpu.sync_copy(h2pad.at[dst_t.at[g]], rbuf)
        pltpu.sync_copy(rbuf, acc_s.at[src_t.at[g]], add=True)
        pltpu.sync_copy(ones_t, cnt_s.at[src_t.at[g]], add=True)
        return 0
    lax.fori_loop(0, WPT, p1, 0)

    plsc.subcore_barrier()

    pltpu.sync_copy(acc_s.at[pl.ds(row0, RPT)], acc_out.at[c, pl.ds(row0, RPT)])
    pltpu.sync_copy(cnt_s.at[pl.ds(row0, RPT)], cnt_out.at[c, pl.ds(row0, RPT)])


def _csl_sc(h2pad, src3d, dst3d):
    mesh = plsc.VectorSubcoreMesh(core_axis_name="c", subcore_axis_name="s")
    f = pl.kernel(
        _csl_sc_body,
        out_type=[
            jax.ShapeDtypeStruct((2, NP, 128), F32),
            jax.ShapeDtypeStruct((2, NP), F32),
        ],
        mesh=mesh,
        compiler_params=pltpu.CompilerParams(needs_layout_passes=False),
        scratch_types=dict(
            src_t=pltpu.VMEM((WPT, CW), I32),
            dst_t=pltpu.VMEM((WPT, CW), I32),
            rbuf=pltpu.VMEM((CW, 128), F32),
            ones_t=pltpu.VMEM((CW,), F32),
            zden_t=pltpu.VMEM((RPT,), F32),
            acc_s=pltpu.VMEM_SHARED((NP, 128), F32),
            cnt_s=pltpu.VMEM_SHARED((NP,), F32),
        ),
    )
    return f(h2pad, src3d, dst3d)


def _tc3_body(a_lo, a_hi, c_lo, c_hi, out_ref):
    cnt = (c_lo[0] + c_hi[0]).reshape(-1)
    inv = 1.0 / jnp.maximum(cnt, 1.0)
    out_ref[...] = (a_lo[0] + a_hi[0]) * inv.reshape(-1, 1)


def _tc3(acc2, cnt2):
    blk = 1024
    lo = lambda i: (0, i, 0)
    hi = lambda i: (1, i, 0)
    cnt3 = cnt2.reshape(2, NP // CW, CW)
    return pl.pallas_call(
        _tc3_body,
        grid=(NP // blk,),
        in_specs=[
            pl.BlockSpec((1, blk, 128), lo),
            pl.BlockSpec((1, blk, 128), hi),
            pl.BlockSpec((1, blk // CW, CW), lo),
            pl.BlockSpec((1, blk // CW, CW), hi),
        ],
        out_specs=pl.BlockSpec((blk, 128), lambda i: (i, 0)),
        out_shape=jax.ShapeDtypeStruct((NP, 128), F32),
    )(acc2, acc2, cnt3, cnt3)


def _pad_edges(idx):
    blocks = idx.reshape(NS, E // NS)
    blocks = jnp.pad(blocks, ((0, 0), (0, NP - E // NS)),
                     constant_values=NP - 1)
    return blocks.reshape(NS, CPT, CW)


def kernel(features, edge_index, edge_CSL, W1, att_src1, att_dst1, W2,
           Wd1, bd1, Wd2, bd2):
    att2p = jnp.zeros((HID, 128), F32)
    att2p = att2p.at[:, 0].set(att_src1).at[:, 1].set(att_dst1)
    a2 = _tc1(features, W1, att2p)
    asrc = jnp.pad(a2[:, 0], (0, NP - N))
    adst = jnp.pad(a2[:, 1], (0, NP - N))

    fcat = jnp.zeros((2 * NP, HALF), F32)
    fcat = fcat.at[0:N].set(features[:, :HALF])
    fcat = fcat.at[NP:NP + N].set(features[:, HALF:])

    srcP = _pad_edges(edge_index[0])
    dstP = _pad_edges(edge_index[1])
    srcN = _pad_edges(edge_CSL[0])
    dstN = _pad_edges(edge_CSL[1])

    aggP, denP = _gat_sc(fcat, asrc, adst, srcP.reshape(-1), dstP)
    aggN, denN = _gat_sc(fcat, asrc, adst, srcN.reshape(-1), dstN)

    h2p, h2np, recp = _tc2(aggP, denP, aggN, denN, W1, W2, Wd1,
                           bd1.reshape(1, HID), Wd2, bd2.reshape(1, IN_DIM))

    acc2, cnt2 = _csl_sc(h2p,
                         srcP.reshape(2 * NS, WPT, CW),
                         dstP.reshape(2 * NS, WPT, CW))
    hp = _tc3(acc2, cnt2)

    return h2p[:N, :OUT], hp[:N, :OUT], h2np[:N, :OUT], recp[:N]


# speedup vs baseline: 9.6046x; 1.2412x over previous
import jax
import jax.numpy as jnp
from jax import lax
from jax.experimental import pallas as pl
from jax.experimental.pallas import tpu as pltpu
from jax.experimental.pallas import tpu_sc as plsc

N = 10000
E = 160000
IN_DIM, HID, OUT = 256, 512, 64
HALF = IN_DIM // 2
NS = 16
NP = 10240
RPT = NP // NS
CW = 128
CPT = NP // CW
WPT = NP // 2 // CW
F32 = jnp.float32
I32 = jnp.int32


def _zvec():
    return jnp.zeros((16,), F32)


def _tc1_body(x_ref, w1_ref, att_ref, out_ref):
    wmat = jnp.dot(w1_ref[...], att_ref[...], preferred_element_type=F32)
    out_ref[...] = jnp.dot(x_ref[...], wmat, preferred_element_type=F32)


def _tc1(features, W1, att2p):
    return pl.pallas_call(
        _tc1_body,
        grid=(25,),
        in_specs=[
            pl.BlockSpec((400, IN_DIM), lambda i: (i, 0)),
            pl.BlockSpec((IN_DIM, HID), lambda i: (0, 0)),
            pl.BlockSpec((HID, 128), lambda i: (0, 0)),
        ],
        out_specs=pl.BlockSpec((400, 128), lambda i: (i, 0)),
        out_shape=jax.ShapeDtypeStruct((N, 128), F32),
    )(features, W1, att2p)


def _gat_sc_body(fcat, asrc_h, adst_h, src_h, dst_h, out, den_out,
                 src_l, dst2, w_l, rbuf, didx2,
                 sem_a, sem_b, sem_c, sem_d, den_s, agg_s):
    c = lax.axis_index("c")
    s = lax.axis_index("s")
    row0 = s * RPT
    EP = NP

    pltpu.sync_copy(src_h.at[pl.ds(s * EP, EP)], src_l)
    pltpu.sync_copy(dst_h.at[s], dst2)

    def zd(i, _):
        w_l[pl.ds(i * 16, 16)] = _zvec()
        return 0
    lax.fori_loop(0, RPT // 16, zd, 0)
    pltpu.sync_copy(w_l.at[pl.ds(0, RPT)], den_s.at[pl.ds(row0, RPT)])

    def zr(i, _):
        for v in range(8):
            rbuf[i, pl.ds(v * 16, 16)] = _zvec()
        return 0
    lax.fori_loop(0, CW, zr, 0)
    for k in range(RPT // CW):
        pltpu.sync_copy(rbuf, agg_s.at[pl.ds(row0 + k * CW, CW)])

    plsc.subcore_barrier()

    def fire1(r, _):
        sl_e = pl.ds(r * CW, CW)
        pltpu.async_copy(asrc_h.at[src_l.at[sl_e]], w_l.at[sl_e], sem_a)
        pltpu.async_copy(adst_h.at[dst2.at[r]], rbuf.at[r], sem_b)
        return 0
    lax.fori_loop(0, CPT, fire1, 0)

    def drain1(r, _):
        pltpu.make_async_copy(asrc_h.at[pl.ds(0, CW)],
                              w_l.at[pl.ds(0, CW)], sem_a).wait()
        pltpu.make_async_copy(adst_h.at[pl.ds(0, CW)], rbuf.at[0], sem_b).wait()
        return 0
    lax.fori_loop(0, CPT, drain1, 0)

    def p1(i, _):
        sl = pl.ds(i * 16, 16)
        e = w_l[sl] + rbuf[i >> 3, pl.ds((i & 7) * 16, 16)]
        e = jnp.where(e > 0.0, e, e * jnp.float32(0.2))
        w_l[sl] = jnp.exp(e)
        return 0
    lax.fori_loop(0, EP // 16, p1, 0)

    def fired(r, _):
        pltpu.async_copy(w_l.at[pl.ds(r * CW, CW)], den_s.at[dst2.at[r]],
                         sem_a, add=True)
        return 0
    lax.fori_loop(0, CPT, fired, 0)

    def draind(r, _):
        pltpu.make_async_copy(w_l.at[pl.ds(0, CW)],
                              den_s.at[dst2.at[0]], sem_a).wait()
        return 0
    lax.fori_loop(0, CPT, draind, 0)

    coff = c * NP

    def padj(i, _):
        sl = pl.ds(i * 16, 16)
        src_l[sl] = src_l[sl] + coff
        return 0
    lax.fori_loop(0, EP // 16, padj, 0)

    plsc.subcore_barrier()

    def _stage_didx(r, half):
        for k in range(4):
            didx2[half, pl.ds(k * 16, 16)] = dst2[r, pl.ds(64 * half + k * 16, 16)]

    def _fire_g(r, half, sem):
        pltpu.async_copy(fcat.at[src_l.at[pl.ds(r * CW + 64 * half, 64)]],
                         rbuf.at[pl.ds(64 * half, 64)], sem)

    def _wait_g(sem):
        pltpu.make_async_copy(fcat.at[pl.ds(0, 64)],
                              rbuf.at[pl.ds(0, 64)], sem).wait()

    def _fire_s(half, sem):
        pltpu.async_copy(rbuf.at[pl.ds(64 * half, 64)],
                         agg_s.at[didx2.at[half]], sem, add=True)

    def _wait_s(half, sem):
        pltpu.make_async_copy(rbuf.at[pl.ds(64 * half, 64)],
                              agg_s.at[didx2.at[half]], sem).wait()

    def _scale(r, half):
        for k in range(4):
            wv = w_l[pl.ds(r * CW + 64 * half + k * 16, 16)]
            for j in range(16):
                wj = wv[j]
                e = 64 * half + k * 16 + j
                for v in range(8):
                    sl = pl.ds(v * 16, 16)
                    rbuf[e, sl] = rbuf[e, sl] * wj

    _stage_didx(0, 0)
    _fire_g(0, 0, sem_a)
    _stage_didx(0, 1)
    _fire_g(0, 1, sem_b)

    def p3(gg, _):
        _wait_g(sem_a)
        _scale(gg, 0)
        _fire_s(0, sem_c)
        _wait_g(sem_b)
        _scale(gg, 1)
        _fire_s(1, sem_d)

        @pl.when(gg < CPT - 1)
        def _():
            _wait_s(0, sem_c)
            _stage_didx(gg + 1, 0)
            _fire_g(gg + 1, 0, sem_a)
            _wait_s(1, sem_d)
            _stage_didx(gg + 1, 1)
            _fire_g(gg + 1, 1, sem_b)
        return 0
    lax.fori_loop(0, CPT, p3, 0)
    _wait_s(0, sem_c)
    _wait_s(1, sem_d)

    plsc.subcore_barrier()

    pltpu.sync_copy(agg_s.at[pl.ds(row0, RPT)], out.at[c, pl.ds(row0, RPT)])
    pltpu.sync_copy(den_s.at[pl.ds(row0, RPT)], den_out.at[c, pl.ds(row0, RPT)])


def _gat_sc(fcat, asrc, adst, srcF, dst3):
    mesh = plsc.VectorSubcoreMesh(core_axis_name="c", subcore_axis_name="s")
    f = pl.kernel(
        _gat_sc_body,
        out_type=[
            jax.ShapeDtypeStruct((2, NP, HALF), F32),
            jax.ShapeDtypeStruct((2, NP), F32),
        ],
        mesh=mesh,
        compiler_params=pltpu.CompilerParams(needs_layout_passes=False),
        scratch_types=dict(
            src_l=pltpu.VMEM((NP,), I32),
            dst2=pltpu.VMEM((CPT, CW), I32),
            w_l=pltpu.VMEM((NP,), F32),
            rbuf=pltpu.VMEM((CW, HALF), F32),
            didx2=pltpu.VMEM((2, 64), I32),
            sem_a=pltpu.SemaphoreType.DMA,
            sem_b=pltpu.SemaphoreType.DMA,
            sem_c=pltpu.SemaphoreType.DMA,
            sem_d=pltpu.SemaphoreType.DMA,
            den_s=pltpu.VMEM_SHARED((NP,), F32),
            agg_s=pltpu.VMEM_SHARED((NP, HALF), F32),
        ),
    )
    return f(fcat, asrc, adst, srcF, dst3)


def _elu(x):
    return jnp.where(x > 0.0, x, jnp.exp(x) - 1.0)


def _tc2_body(apl_ref, aph_ref, anl_ref, anh_ref, dp_ref, dn_ref,
              w1_ref, w2_ref, wd1_ref, bd1_ref, wd2_ref, bd2_ref,
              h2_ref, h2n_ref, rec_ref):
    w1l = w1_ref[0:HALF, :]
    w1h = w1_ref[HALF:IN_DIM, :]
    ivp = (1.0 / (dp_ref[...].reshape(-1) + jnp.float32(1e-16))).reshape(-1, 1)
    ivn = (1.0 / (dn_ref[...].reshape(-1) + jnp.float32(1e-16))).reshape(-1, 1)
    h1 = jnp.dot(apl_ref[0] * ivp, w1l, preferred_element_type=F32)
    h1 = h1 + jnp.dot(aph_ref[0] * ivp, w1h, preferred_element_type=F32)
    h2 = jnp.dot(_elu(h1), w2_ref[...], preferred_element_type=F32)
    h1n = jnp.dot(anl_ref[0] * ivn, w1l, preferred_element_type=F32)
    h1n = h1n + jnp.dot(anh_ref[0] * ivn, w1h, preferred_element_type=F32)
    h2n = jnp.dot(_elu(h1n), w2_ref[...], preferred_element_type=F32)
    r1 = _elu(jnp.dot(h2, wd1_ref[...], preferred_element_type=F32)
              + bd1_ref[...])
    rec = jnp.dot(r1, wd2_ref[...], preferred_element_type=F32) + bd2_ref[...]
    zpad = jnp.zeros((h2.shape[0], 128 - OUT), F32)
    h2_ref[...] = jnp.concatenate([h2, zpad], axis=1)
    h2n_ref[...] = jnp.concatenate([h2n, zpad], axis=1)
    rec_ref[...] = rec


def _tc2(aggP, denP, aggN, denN, W1, W2, Wd1, bd1r, Wd2, bd2r):
    blk = 1024
    lo = lambda i: (0, i, 0)
    hi = lambda i: (1, i, 0)
    dP = denP[0].reshape(NP // CW, CW)
    dN = denN[0].reshape(NP // CW, CW)

    def full(shape):
        return pl.BlockSpec(shape, lambda i: tuple(0 for _ in shape))

    return pl.pallas_call(
        _tc2_body,
        grid=(NP // blk,),
        in_specs=[
            pl.BlockSpec((1, blk, HALF), lo),
            pl.BlockSpec((1, blk, HALF), hi),
            pl.BlockSpec((1, blk, HALF), lo),
            pl.BlockSpec((1, blk, HALF), hi),
            pl.BlockSpec((blk // CW, CW), lambda i: (i, 0)),
            pl.BlockSpec((blk // CW, CW), lambda i: (i, 0)),
            full((IN_DIM, HID)),
            full((HID, OUT)),
            full((OUT, HID)),
            full((1, HID)),
            full((HID, IN_DIM)),
            full((1, IN_DIM)),
        ],
        out_specs=[
            pl.BlockSpec((blk, 128), lambda i: (i, 0)),
            pl.BlockSpec((blk, 128), lambda i: (i, 0)),
            pl.BlockSpec((blk, IN_DIM), lambda i: (i, 0)),
        ],
        out_shape=[
            jax.ShapeDtypeStruct((NP, 128), F32),
            jax.ShapeDtypeStruct((NP, 128), F32),
            jax.ShapeDtypeStruct((NP, IN_DIM), F32),
        ],
    )(aggP, aggP, aggN, aggN, dP, dN, W1, W2, Wd1, bd1r, Wd2, bd2r)


def _csl_sc_body(h2pad, src_h, dst_h, acc_out, cnt_out,
                 src_t, dst_t, rbuf, ones_t, zden_t, acc_s, cnt_s):
    c = lax.axis_index("c")
    s = lax.axis_index("s")
    row0 = s * RPT
    w = c * NS + s

    pltpu.sync_copy(src_h.at[w], src_t)
    pltpu.sync_copy(dst_h.at[w], dst_t)

    def zd(i, _):
        zden_t[pl.ds(i * 16, 16)] = _zvec()
        return 0
    lax.fori_loop(0, RPT // 16, zd, 0)
    pltpu.sync_copy(zden_t, cnt_s.at[pl.ds(row0, RPT)])

    def zr(i, _):
        for v in range(128 // 16):
            rbuf[i, pl.ds(v * 16, 16)] = _zvec()
        return 0
    lax.fori_loop(0, CW, zr, 0)
    for k in range(RPT // CW):
        pltpu.sync_copy(rbuf, acc_s.at[pl.ds(row0 + k * CW, CW)])

    for k in range(CW // 16):
        ones_t[pl.ds(k * 16, 16)] = jnp.ones((16,), F32)

    plsc.subcore_barrier()

    def p1(g, _):
        pltpu.sync_copy(h2pad.at[dst_t.at[g]], rbuf)
        pltpu.sync_copy(rbuf, acc_s.at[src_t.at[g]], add=True)
        pltpu.sync_copy(ones_t, cnt_s.at[src_t.at[g]], add=True)
        return 0
    lax.fori_loop(0, WPT, p1, 0)

    plsc.subcore_barrier()

    pltpu.sync_copy(acc_s.at[pl.ds(row0, RPT)], acc_out.at[c, pl.ds(row0, RPT)])
    pltpu.sync_copy(cnt_s.at[pl.ds(row0, RPT)], cnt_out.at[c, pl.ds(row0, RPT)])


def _csl_sc(h2pad, src3d, dst3d):
    mesh = plsc.VectorSubcoreMesh(core_axis_name="c", subcore_axis_name="s")
    f = pl.kernel(
        _csl_sc_body,
        out_type=[
            jax.ShapeDtypeStruct((2, NP, 128), F32),
            jax.ShapeDtypeStruct((2, NP), F32),
        ],
        mesh=mesh,
        compiler_params=pltpu.CompilerParams(needs_layout_passes=False),
        scratch_types=dict(
            src_t=pltpu.VMEM((WPT, CW), I32),
            dst_t=pltpu.VMEM((WPT, CW), I32),
            rbuf=pltpu.VMEM((CW, 128), F32),
            ones_t=pltpu.VMEM((CW,), F32),
            zden_t=pltpu.VMEM((RPT,), F32),
            acc_s=pltpu.VMEM_SHARED((NP, 128), F32),
            cnt_s=pltpu.VMEM_SHARED((NP,), F32),
        ),
    )
    return f(h2pad, src3d, dst3d)


def _tc3_body(a_lo, a_hi, c_lo, c_hi, out_ref):
    cnt = (c_lo[0] + c_hi[0]).reshape(-1)
    inv = 1.0 / jnp.maximum(cnt, 1.0)
    out_ref[...] = (a_lo[0] + a_hi[0]) * inv.reshape(-1, 1)


def _tc3(acc2, cnt2):
    blk = 1024
    lo = lambda i: (0, i, 0)
    hi = lambda i: (1, i, 0)
    cnt3 = cnt2.reshape(2, NP // CW, CW)
    return pl.pallas_call(
        _tc3_body,
        grid=(NP // blk,),
        in_specs=[
            pl.BlockSpec((1, blk, 128), lo),
            pl.BlockSpec((1, blk, 128), hi),
            pl.BlockSpec((1, blk // CW, CW), lo),
            pl.BlockSpec((1, blk // CW, CW), hi),
        ],
        out_specs=pl.BlockSpec((blk, 128), lambda i: (i, 0)),
        out_shape=jax.ShapeDtypeStruct((NP, 128), F32),
    )(acc2, acc2, cnt3, cnt3)


def _pad_edges(idx):
    blocks = idx.reshape(NS, E // NS)
    blocks = jnp.pad(blocks, ((0, 0), (0, NP - E // NS)),
                     constant_values=NP - 1)
    return blocks.reshape(NS, CPT, CW)


def kernel(features, edge_index, edge_CSL, W1, att_src1, att_dst1, W2,
           Wd1, bd1, Wd2, bd2):
    att2p = jnp.zeros((HID, 128), F32)
    att2p = att2p.at[:, 0].set(att_src1).at[:, 1].set(att_dst1)
    a2 = _tc1(features, W1, att2p)
    asrc = jnp.pad(a2[:, 0], (0, NP - N))
    adst = jnp.pad(a2[:, 1], (0, NP - N))

    fcat = jnp.zeros((2 * NP, HALF), F32)
    fcat = fcat.at[0:N].set(features[:, :HALF])
    fcat = fcat.at[NP:NP + N].set(features[:, HALF:])

    srcP = _pad_edges(edge_index[0])
    dstP = _pad_edges(edge_index[1])
    srcN = _pad_edges(edge_CSL[0])
    dstN = _pad_edges(edge_CSL[1])

    aggP, denP = _gat_sc(fcat, asrc, adst, srcP.reshape(-1), dstP)
    aggN, denN = _gat_sc(fcat, asrc, adst, srcN.reshape(-1), dstN)

    h2p, h2np, recp = _tc2(aggP, denP, aggN, denN, W1, W2, Wd1,
                           bd1.reshape(1, HID), Wd2, bd2.reshape(1, IN_DIM))

    acc2, cnt2 = _csl_sc(h2p,
                         srcP.reshape(2 * NS, WPT, CW),
                         dstP.reshape(2 * NS, WPT, CW))
    hp = _tc3(acc2, cnt2)

    return h2p[:N, :OUT], hp[:N, :OUT], h2np[:N, :OUT], recp[:N]


# speedup vs baseline: 9.7045x; 1.0104x over previous
import jax
import jax.numpy as jnp
from jax import lax
from jax.experimental import pallas as pl
from jax.experimental.pallas import tpu as pltpu
from jax.experimental.pallas import tpu_sc as plsc

N = 10000
E = 160000
IN_DIM, HID, OUT = 256, 512, 64
HALF = IN_DIM // 2
NS = 16
NP = 10240
RPT = NP // NS
CW = 128
CPT = NP // CW
WPT = NP // 2 // CW
F32 = jnp.float32
I32 = jnp.int32


def _zvec():
    return jnp.zeros((16,), F32)


def _tc1_body(x_ref, w1_ref, att_ref, out_ref):
    wmat = jnp.dot(w1_ref[...], att_ref[...], preferred_element_type=F32)
    out_ref[...] = jnp.dot(x_ref[...], wmat, preferred_element_type=F32)


def _tc1(features, W1, att2p):
    return pl.pallas_call(
        _tc1_body,
        grid=(25,),
        in_specs=[
            pl.BlockSpec((400, IN_DIM), lambda i: (i, 0)),
            pl.BlockSpec((IN_DIM, HID), lambda i: (0, 0)),
            pl.BlockSpec((HID, 128), lambda i: (0, 0)),
        ],
        out_specs=pl.BlockSpec((400, 128), lambda i: (i, 0)),
        out_shape=jax.ShapeDtypeStruct((N, 128), F32),
    )(features, W1, att2p)


def _gat_sc_body(fcat, asrc_h, adst_h, src_h, dst_h, out, den_out,
                 src_l, dst2, w_l, rbuf, didx2,
                 sem_a, sem_b, sem_c, sem_d, den_s, agg_s):
    c = lax.axis_index("c")
    s = lax.axis_index("s")
    row0 = s * RPT
    EP = NP

    pltpu.sync_copy(src_h.at[pl.ds(s * EP, EP)], src_l)
    pltpu.sync_copy(dst_h.at[s], dst2)

    def zd(i, _):
        w_l[pl.ds(i * 16, 16)] = _zvec()
        return 0
    lax.fori_loop(0, RPT // 16, zd, 0)
    pltpu.sync_copy(w_l.at[pl.ds(0, RPT)], den_s.at[pl.ds(row0, RPT)])

    def zr(i, _):
        for v in range(8):
            rbuf[i, pl.ds(v * 16, 16)] = _zvec()
        return 0
    lax.fori_loop(0, CW, zr, 0)
    for k in range(RPT // CW):
        pltpu.sync_copy(rbuf, agg_s.at[pl.ds(row0 + k * CW, CW)])

    plsc.subcore_barrier()

    def fire1(r, _):
        sl_e = pl.ds(r * CW, CW)
        pltpu.async_copy(asrc_h.at[src_l.at[sl_e]], w_l.at[sl_e], sem_a)
        pltpu.async_copy(adst_h.at[dst2.at[r]], rbuf.at[r], sem_b)
        return 0
    lax.fori_loop(0, CPT, fire1, 0)

    def drain1(r, _):
        pltpu.make_async_copy(asrc_h.at[pl.ds(0, CW)],
                              w_l.at[pl.ds(0, CW)], sem_a).wait()
        pltpu.make_async_copy(adst_h.at[pl.ds(0, CW)], rbuf.at[0], sem_b).wait()
        return 0
    lax.fori_loop(0, CPT, drain1, 0)

    def p1(i, _):
        sl = pl.ds(i * 16, 16)
        e = w_l[sl] + rbuf[i >> 3, pl.ds((i & 7) * 16, 16)]
        e = jnp.where(e > 0.0, e, e * jnp.float32(0.2))
        w_l[sl] = jnp.exp(e)
        return 0
    lax.fori_loop(0, EP // 16, p1, 0)

    def fired(r, _):
        pltpu.async_copy(w_l.at[pl.ds(r * CW, CW)], den_s.at[dst2.at[r]],
                         sem_a, add=True)
        return 0
    lax.fori_loop(0, CPT, fired, 0)

    def draind(r, _):
        pltpu.make_async_copy(w_l.at[pl.ds(0, CW)],
                              den_s.at[dst2.at[0]], sem_a).wait()
        return 0
    lax.fori_loop(0, CPT, draind, 0)

    coff = c * NP

    def padj(i, _):
        sl = pl.ds(i * 16, 16)
        src_l[sl] = src_l[sl] + coff
        return 0
    lax.fori_loop(0, EP // 16, padj, 0)

    plsc.subcore_barrier()

    def _stage_didx(r, half):
        for k in range(4):
            didx2[half, pl.ds(k * 16, 16)] = dst2[r, pl.ds(64 * half + k * 16, 16)]

    def _fire_g(r, half, sem):
        pltpu.async_copy(fcat.at[src_l.at[pl.ds(r * CW + 64 * half, 64)]],
                         rbuf.at[pl.ds(64 * half, 64)], sem)

    def _wait_g(sem):
        pltpu.make_async_copy(fcat.at[pl.ds(0, 64)],
                              rbuf.at[pl.ds(0, 64)], sem).wait()

    def _fire_s(half, sem):
        pltpu.async_copy(rbuf.at[pl.ds(64 * half, 64)],
                         agg_s.at[didx2.at[half]], sem, add=True)

    def _wait_s(half, sem):
        pltpu.make_async_copy(rbuf.at[pl.ds(64 * half, 64)],
                              agg_s.at[didx2.at[half]], sem).wait()

    def _scale(r, half):
        for k in range(4):
            wv = w_l[pl.ds(r * CW + 64 * half + k * 16, 16)]
            for j in range(16):
                wj = wv[j]
                e = 64 * half + k * 16 + j
                for v in range(8):
                    sl = pl.ds(v * 16, 16)
                    rbuf[e, sl] = rbuf[e, sl] * wj

    _stage_didx(0, 0)
    _fire_g(0, 0, sem_a)
    _stage_didx(0, 1)
    _fire_g(0, 1, sem_b)

    def p3(gg, _):
        _wait_g(sem_a)
        _scale(gg, 0)
        _fire_s(0, sem_c)
        _wait_g(sem_b)
        _scale(gg, 1)
        _fire_s(1, sem_d)

        @pl.when(gg < CPT - 1)
        def _():
            _wait_s(0, sem_c)
            _stage_didx(gg + 1, 0)
            _fire_g(gg + 1, 0, sem_a)
            _wait_s(1, sem_d)
            _stage_didx(gg + 1, 1)
            _fire_g(gg + 1, 1, sem_b)
        return 0
    lax.fori_loop(0, CPT, p3, 0)
    _wait_s(0, sem_c)
    _wait_s(1, sem_d)

    plsc.subcore_barrier()

    pltpu.sync_copy(agg_s.at[pl.ds(row0, RPT)], out.at[c, pl.ds(row0, RPT)])
    pltpu.sync_copy(den_s.at[pl.ds(row0, RPT)], den_out.at[c, pl.ds(row0, RPT)])


def _gat_sc(fcat, asrc, adst, srcF, dst3):
    mesh = plsc.VectorSubcoreMesh(core_axis_name="c", subcore_axis_name="s")
    f = pl.kernel(
        _gat_sc_body,
        out_type=[
            jax.ShapeDtypeStruct((2, NP, HALF), F32),
            jax.ShapeDtypeStruct((2, NP), F32),
        ],
        mesh=mesh,
        compiler_params=pltpu.CompilerParams(needs_layout_passes=False),
        scratch_types=dict(
            src_l=pltpu.VMEM((NP,), I32),
            dst2=pltpu.VMEM((CPT, CW), I32),
            w_l=pltpu.VMEM((NP,), F32),
            rbuf=pltpu.VMEM((CW, HALF), F32),
            didx2=pltpu.VMEM((2, 64), I32),
            sem_a=pltpu.SemaphoreType.DMA,
            sem_b=pltpu.SemaphoreType.DMA,
            sem_c=pltpu.SemaphoreType.DMA,
            sem_d=pltpu.SemaphoreType.DMA,
            den_s=pltpu.VMEM_SHARED((NP,), F32),
            agg_s=pltpu.VMEM_SHARED((NP, HALF), F32),
        ),
    )
    return f(fcat, asrc, adst, srcF, dst3)


def _elu(x):
    return jnp.where(x > 0.0, x, jnp.exp(x) - 1.0)


def _tc2_body(apl_ref, aph_ref, anl_ref, anh_ref, dp_ref, dn_ref,
              w1_ref, w2_ref, wd1_ref, bd1_ref, wd2_ref, bd2_ref,
              h2_ref, h2n_ref, rec_ref):
    w1l = w1_ref[0:HALF, :]
    w1h = w1_ref[HALF:IN_DIM, :]
    ivp = (1.0 / (dp_ref[...].reshape(-1) + jnp.float32(1e-16))).reshape(-1, 1)
    ivn = (1.0 / (dn_ref[...].reshape(-1) + jnp.float32(1e-16))).reshape(-1, 1)
    h1 = jnp.dot(apl_ref[0] * ivp, w1l, preferred_element_type=F32)
    h1 = h1 + jnp.dot(aph_ref[0] * ivp, w1h, preferred_element_type=F32)
    h2 = jnp.dot(_elu(h1), w2_ref[...], preferred_element_type=F32)
    h1n = jnp.dot(anl_ref[0] * ivn, w1l, preferred_element_type=F32)
    h1n = h1n + jnp.dot(anh_ref[0] * ivn, w1h, preferred_element_type=F32)
    h2n = jnp.dot(_elu(h1n), w2_ref[...], preferred_element_type=F32)
    r1 = _elu(jnp.dot(h2, wd1_ref[...], preferred_element_type=F32)
              + bd1_ref[...])
    rec = jnp.dot(r1, wd2_ref[...], preferred_element_type=F32) + bd2_ref[...]
    zpad = jnp.zeros((h2.shape[0], 128 - OUT), F32)
    h2_ref[...] = jnp.concatenate([h2, zpad], axis=1)
    h2n_ref[...] = jnp.concatenate([h2n, zpad], axis=1)
    rec_ref[...] = rec


def _tc2(aggP, denP, aggN, denN, W1, W2, Wd1, bd1r, Wd2, bd2r):
    blk = 1024
    lo = lambda i: (0, i, 0)
    hi = lambda i: (1, i, 0)
    dP = denP[0].reshape(NP // CW, CW)
    dN = denN[0].reshape(NP // CW, CW)

    def full(shape):
        return pl.BlockSpec(shape, lambda i: tuple(0 for _ in shape))

    return pl.pallas_call(
        _tc2_body,
        grid=(NP // blk,),
        in_specs=[
            pl.BlockSpec((1, blk, HALF), lo),
            pl.BlockSpec((1, blk, HALF), hi),
            pl.BlockSpec((1, blk, HALF), lo),
            pl.BlockSpec((1, blk, HALF), hi),
            pl.BlockSpec((blk // CW, CW), lambda i: (i, 0)),
            pl.BlockSpec((blk // CW, CW), lambda i: (i, 0)),
            full((IN_DIM, HID)),
            full((HID, OUT)),
            full((OUT, HID)),
            full((1, HID)),
            full((HID, IN_DIM)),
            full((1, IN_DIM)),
        ],
        out_specs=[
            pl.BlockSpec((blk, 128), lambda i: (i, 0)),
            pl.BlockSpec((blk, 128), lambda i: (i, 0)),
            pl.BlockSpec((blk, IN_DIM), lambda i: (i, 0)),
        ],
        out_shape=[
            jax.ShapeDtypeStruct((NP, 128), F32),
            jax.ShapeDtypeStruct((NP, 128), F32),
            jax.ShapeDtypeStruct((NP, IN_DIM), F32),
        ],
    )(aggP, aggP, aggN, aggN, dP, dN, W1, W2, Wd1, bd1r, Wd2, bd2r)


def _csl_sc_body(h2pad, src_h, dst_h, acc_out, cnt_out,
                 src2, dst_l, rbuf, ones_t, zden_t, didx2,
                 sem_a, sem_b, sem_c, sem_d, sem_e, acc_s, cnt_s):
    c = lax.axis_index("c")
    s = lax.axis_index("s")
    row0 = s * RPT
    w = c * NS + s
    EPW = NP // 2

    pltpu.sync_copy(src_h.at[w], src2)
    pltpu.sync_copy(dst_h.at[pl.ds(w * EPW, EPW)], dst_l)

    def zd(i, _):
        zden_t[pl.ds(i * 16, 16)] = _zvec()
        return 0
    lax.fori_loop(0, RPT // 16, zd, 0)
    pltpu.sync_copy(zden_t, cnt_s.at[pl.ds(row0, RPT)])

    def zr(i, _):
        for v in range(128 // 16):
            rbuf[i, pl.ds(v * 16, 16)] = _zvec()
        return 0
    lax.fori_loop(0, CW, zr, 0)
    for k in range(RPT // CW):
        pltpu.sync_copy(rbuf, acc_s.at[pl.ds(row0 + k * CW, CW)])

    for k in range(CW // 16):
        ones_t[pl.ds(k * 16, 16)] = jnp.ones((16,), F32)

    plsc.subcore_barrier()

    def firec(r, _):
        pltpu.async_copy(ones_t, cnt_s.at[src2.at[r]], sem_e, add=True)
        return 0
    lax.fori_loop(0, WPT, firec, 0)

    def _stage_didx(r, half):
        for k in range(4):
            didx2[half, pl.ds(k * 16, 16)] = src2[r, pl.ds(64 * half + k * 16, 16)]

    def _fire_g(g, half, sem):
        pltpu.async_copy(h2pad.at[dst_l.at[pl.ds(g * 64, 64)]],
                         rbuf.at[pl.ds(64 * half, 64)], sem)

    def _wait_g(sem):
        pltpu.make_async_copy(h2pad.at[pl.ds(0, 64)],
                              rbuf.at[pl.ds(0, 64)], sem).wait()

    def _fire_s(half, sem):
        pltpu.async_copy(rbuf.at[pl.ds(64 * half, 64)],
                         acc_s.at[didx2.at[half]], sem, add=True)

    def _wait_s(half, sem):
        pltpu.make_async_copy(rbuf.at[pl.ds(64 * half, 64)],
                              acc_s.at[didx2.at[half]], sem).wait()

    _stage_didx(0, 0)
    _fire_g(0, 0, sem_a)
    _stage_didx(0, 1)
    _fire_g(1, 1, sem_b)

    def p1(gg, _):
        _wait_g(sem_a)
        _fire_s(0, sem_c)
        _wait_g(sem_b)
        _fire_s(1, sem_d)

        @pl.when(gg < WPT - 1)
        def _():
            _wait_s(0, sem_c)
            _stage_didx(gg + 1, 0)
            _fire_g(2 * gg + 2, 0, sem_a)
            _wait_s(1, sem_d)
            _stage_didx(gg + 1, 1)
            _fire_g(2 * gg + 3, 1, sem_b)
        return 0
    lax.fori_loop(0, WPT, p1, 0)
    _wait_s(0, sem_c)
    _wait_s(1, sem_d)

    def drainc(r, _):
        pltpu.make_async_copy(ones_t, cnt_s.at[src2.at[0]], sem_e).wait()
        return 0
    lax.fori_loop(0, WPT, drainc, 0)

    plsc.subcore_barrier()

    pltpu.sync_copy(acc_s.at[pl.ds(row0, RPT)], acc_out.at[c, pl.ds(row0, RPT)])
    pltpu.sync_copy(cnt_s.at[pl.ds(row0, RPT)], cnt_out.at[c, pl.ds(row0, RPT)])


def _csl_sc(h2pad, src3d, dstF):
    mesh = plsc.VectorSubcoreMesh(core_axis_name="c", subcore_axis_name="s")
    f = pl.kernel(
        _csl_sc_body,
        out_type=[
            jax.ShapeDtypeStruct((2, NP, 128), F32),
            jax.ShapeDtypeStruct((2, NP), F32),
        ],
        mesh=mesh,
        compiler_params=pltpu.CompilerParams(needs_layout_passes=False),
        scratch_types=dict(
            src2=pltpu.VMEM((WPT, CW), I32),
            dst_l=pltpu.VMEM((NP // 2,), I32),
            rbuf=pltpu.VMEM((CW, 128), F32),
            ones_t=pltpu.VMEM((CW,), F32),
            zden_t=pltpu.VMEM((RPT,), F32),
            didx2=pltpu.VMEM((2, 64), I32),
            sem_a=pltpu.SemaphoreType.DMA,
            sem_b=pltpu.SemaphoreType.DMA,
            sem_c=pltpu.SemaphoreType.DMA,
            sem_d=pltpu.SemaphoreType.DMA,
            sem_e=pltpu.SemaphoreType.DMA,
            acc_s=pltpu.VMEM_SHARED((NP, 128), F32),
            cnt_s=pltpu.VMEM_SHARED((NP,), F32),
        ),
    )
    return f(h2pad, src3d, dstF)


def _tc3_body(a_lo, a_hi, c_lo, c_hi, out_ref):
    cnt = (c_lo[0] + c_hi[0]).reshape(-1)
    inv = 1.0 / jnp.maximum(cnt, 1.0)
    out_ref[...] = (a_lo[0] + a_hi[0]) * inv.reshape(-1, 1)


def _tc3(acc2, cnt2):
    blk = 1024
    lo = lambda i: (0, i, 0)
    hi = lambda i: (1, i, 0)
    cnt3 = cnt2.reshape(2, NP // CW, CW)
    return pl.pallas_call(
        _tc3_body,
        grid=(NP // blk,),
        in_specs=[
            pl.BlockSpec((1, blk, 128), lo),
            pl.BlockSpec((1, blk, 128), hi),
            pl.BlockSpec((1, blk // CW, CW), lo),
            pl.BlockSpec((1, blk // CW, CW), hi),
        ],
        out_specs=pl.BlockSpec((blk, 128), lambda i: (i, 0)),
        out_shape=jax.ShapeDtypeStruct((NP, 128), F32),
    )(acc2, acc2, cnt3, cnt3)


def _pad_edges(idx):
    blocks = idx.reshape(NS, E // NS)
    blocks = jnp.pad(blocks, ((0, 0), (0, NP - E // NS)),
                     constant_values=NP - 1)
    return blocks.reshape(NS, CPT, CW)


def kernel(features, edge_index, edge_CSL, W1, att_src1, att_dst1, W2,
           Wd1, bd1, Wd2, bd2):
    att2p = jnp.zeros((HID, 128), F32)
    att2p = att2p.at[:, 0].set(att_src1).at[:, 1].set(att_dst1)
    a2 = _tc1(features, W1, att2p)
    asrc = jnp.pad(a2[:, 0], (0, NP - N))
    adst = jnp.pad(a2[:, 1], (0, NP - N))

    fcat = jnp.zeros((2 * NP, HALF), F32)
    fcat = fcat.at[0:N].set(features[:, :HALF])
    fcat = fcat.at[NP:NP + N].set(features[:, HALF:])

    srcP = _pad_edges(edge_index[0])
    dstP = _pad_edges(edge_index[1])
    srcN = _pad_edges(edge_CSL[0])
    dstN = _pad_edges(edge_CSL[1])

    aggP, denP = _gat_sc(fcat, asrc, adst, srcP.reshape(-1), dstP)
    aggN, denN = _gat_sc(fcat, asrc, adst, srcN.reshape(-1), dstN)

    h2p, h2np, recp = _tc2(aggP, denP, aggN, denN, W1, W2, Wd1,
                           bd1.reshape(1, HID), Wd2, bd2.reshape(1, IN_DIM))

    acc2, cnt2 = _csl_sc(h2p, srcP.reshape(2 * NS, WPT, CW),
                         dstP.reshape(-1))
    hp = _tc3(acc2, cnt2)

    return h2p[:N, :OUT], hp[:N, :OUT], h2np[:N, :OUT], recp[:N]


# speedup vs baseline: 9.7576x; 1.0055x over previous
import jax
import jax.numpy as jnp
from jax import lax
from jax.experimental import pallas as pl
from jax.experimental.pallas import tpu as pltpu
from jax.experimental.pallas import tpu_sc as plsc

N = 10000
E = 160000
IN_DIM, HID, OUT = 256, 512, 64
HALF = IN_DIM // 2
NS = 16
NP = 10240
RPT = NP // NS
CW = 128
CPT = NP // CW
WPT = NP // 2 // CW
F32 = jnp.float32
I32 = jnp.int32


def _zvec():
    return jnp.zeros((16,), F32)


def _tc1_body(x_ref, w1_ref, att_ref, out_ref):
    wmat = jnp.dot(w1_ref[...], att_ref[...], preferred_element_type=F32)
    out_ref[...] = jnp.dot(x_ref[...], wmat, preferred_element_type=F32)


def _tc1(features, W1, att2p):
    return pl.pallas_call(
        _tc1_body,
        grid=(25,),
        in_specs=[
            pl.BlockSpec((400, IN_DIM), lambda i: (i, 0)),
            pl.BlockSpec((IN_DIM, HID), lambda i: (0, 0)),
            pl.BlockSpec((HID, 128), lambda i: (0, 0)),
        ],
        out_specs=pl.BlockSpec((400, 128), lambda i: (i, 0)),
        out_shape=jax.ShapeDtypeStruct((N, 128), F32),
    )(features, W1, att2p)


def _gat_sc_body(fcat, asrc_h, adst_h, src_h, dst_h, out, den_out,
                 src_l, dst2, w_l, rbuf, didx2,
                 sem_a, sem_b, sem_c, sem_d, den_s, agg_s):
    c = lax.axis_index("c")
    s = lax.axis_index("s")
    row0 = s * RPT
    EP = NP

    pltpu.sync_copy(src_h.at[pl.ds(s * EP, EP)], src_l)
    pltpu.sync_copy(dst_h.at[s], dst2)

    def zd(i, _):
        w_l[pl.ds(i * 16, 16)] = _zvec()
        return 0
    lax.fori_loop(0, RPT // 16, zd, 0)
    pltpu.sync_copy(w_l.at[pl.ds(0, RPT)], den_s.at[pl.ds(row0, RPT)])

    def zr(i, _):
        for v in range(8):
            rbuf[i, pl.ds(v * 16, 16)] = _zvec()
        return 0
    lax.fori_loop(0, CW, zr, 0)
    for k in range(RPT // CW):
        pltpu.sync_copy(rbuf, agg_s.at[pl.ds(row0 + k * CW, CW)])

    plsc.subcore_barrier()

    def fire1(r, _):
        sl_e = pl.ds(r * CW, CW)
        pltpu.async_copy(asrc_h.at[src_l.at[sl_e]], w_l.at[sl_e], sem_a)
        pltpu.async_copy(adst_h.at[dst2.at[r]], rbuf.at[r], sem_b)

        @pl.when(r >= 8)
        def _():
            pltpu.make_async_copy(asrc_h.at[pl.ds(0, CW)],
                                  w_l.at[pl.ds(0, CW)], sem_a).wait()
            pltpu.make_async_copy(adst_h.at[pl.ds(0, CW)],
                                  rbuf.at[0], sem_b).wait()
        return 0
    lax.fori_loop(0, CPT, fire1, 0)

    def drain1(r, _):
        pltpu.make_async_copy(asrc_h.at[pl.ds(0, CW)],
                              w_l.at[pl.ds(0, CW)], sem_a).wait()
        pltpu.make_async_copy(adst_h.at[pl.ds(0, CW)], rbuf.at[0], sem_b).wait()
        return 0
    lax.fori_loop(0, 8, drain1, 0)

    def p1(i, _):
        sl = pl.ds(i * 16, 16)
        e = w_l[sl] + rbuf[i >> 3, pl.ds((i & 7) * 16, 16)]
        e = jnp.where(e > 0.0, e, e * jnp.float32(0.2))
        w_l[sl] = jnp.exp(e)
        return 0
    lax.fori_loop(0, EP // 16, p1, 0)

    def fired(r, _):
        pltpu.async_copy(w_l.at[pl.ds(r * CW, CW)], den_s.at[dst2.at[r]],
                         sem_a, add=True)

        @pl.when(r >= 8)
        def _():
            pltpu.make_async_copy(w_l.at[pl.ds(0, CW)],
                                  den_s.at[dst2.at[0]], sem_a).wait()
        return 0
    lax.fori_loop(0, CPT, fired, 0)

    def draind(r, _):
        pltpu.make_async_copy(w_l.at[pl.ds(0, CW)],
                              den_s.at[dst2.at[0]], sem_a).wait()
        return 0
    lax.fori_loop(0, 8, draind, 0)

    coff = c * NP

    def padj(i, _):
        sl = pl.ds(i * 16, 16)
        src_l[sl] = src_l[sl] + coff
        return 0
    lax.fori_loop(0, EP // 16, padj, 0)

    plsc.subcore_barrier()

    def _stage_didx(r, half):
        for k in range(4):
            didx2[half, pl.ds(k * 16, 16)] = dst2[r, pl.ds(64 * half + k * 16, 16)]

    def _fire_g(r, half, sem):
        pltpu.async_copy(fcat.at[src_l.at[pl.ds(r * CW + 64 * half, 64)]],
                         rbuf.at[pl.ds(64 * half, 64)], sem)

    def _wait_g(sem):
        pltpu.make_async_copy(fcat.at[pl.ds(0, 64)],
                              rbuf.at[pl.ds(0, 64)], sem).wait()

    def _fire_s(half, sem):
        pltpu.async_copy(rbuf.at[pl.ds(64 * half, 64)],
                         agg_s.at[didx2.at[half]], sem, add=True)

    def _wait_s(half, sem):
        pltpu.make_async_copy(rbuf.at[pl.ds(64 * half, 64)],
                              agg_s.at[didx2.at[half]], sem).wait()

    def _scale(r, half):
        for k in range(4):
            wv = w_l[pl.ds(r * CW + 64 * half + k * 16, 16)]
            for j in range(16):
                wj = wv[j]
                e = 64 * half + k * 16 + j
                for v in range(8):
                    sl = pl.ds(v * 16, 16)
                    rbuf[e, sl] = rbuf[e, sl] * wj

    _stage_didx(0, 0)
    _fire_g(0, 0, sem_a)
    _stage_didx(0, 1)
    _fire_g(0, 1, sem_b)

    def p3(gg, _):
        _wait_g(sem_a)
        _scale(gg, 0)
        _fire_s(0, sem_c)
        _wait_g(sem_b)
        _scale(gg, 1)
        _fire_s(1, sem_d)

        @pl.when(gg < CPT - 1)
        def _():
            _wait_s(0, sem_c)
            _stage_didx(gg + 1, 0)
            _fire_g(gg + 1, 0, sem_a)
            _wait_s(1, sem_d)
            _stage_didx(gg + 1, 1)
            _fire_g(gg + 1, 1, sem_b)
        return 0
    lax.fori_loop(0, CPT, p3, 0)
    _wait_s(0, sem_c)
    _wait_s(1, sem_d)

    plsc.subcore_barrier()

    pltpu.sync_copy(agg_s.at[pl.ds(row0, RPT)], out.at[c, pl.ds(row0, RPT)])
    pltpu.sync_copy(den_s.at[pl.ds(row0, RPT)], den_out.at[c, pl.ds(row0, RPT)])


def _gat_sc(fcat, asrc, adst, srcF, dst3):
    mesh = plsc.VectorSubcoreMesh(core_axis_name="c", subcore_axis_name="s")
    f = pl.kernel(
        _gat_sc_body,
        out_type=[
            jax.ShapeDtypeStruct((2, NP, HALF), F32),
            jax.ShapeDtypeStruct((2, NP), F32),
        ],
        mesh=mesh,
        compiler_params=pltpu.CompilerParams(needs_layout_passes=False),
        scratch_types=dict(
            src_l=pltpu.VMEM((NP,), I32),
            dst2=pltpu.VMEM((CPT, CW), I32),
            w_l=pltpu.VMEM((NP,), F32),
            rbuf=pltpu.VMEM((CW, HALF), F32),
            didx2=pltpu.VMEM((2, 64), I32),
            sem_a=pltpu.SemaphoreType.DMA,
            sem_b=pltpu.SemaphoreType.DMA,
            sem_c=pltpu.SemaphoreType.DMA,
            sem_d=pltpu.SemaphoreType.DMA,
            den_s=pltpu.VMEM_SHARED((NP,), F32),
            agg_s=pltpu.VMEM_SHARED((NP, HALF), F32),
        ),
    )
    return f(fcat, asrc, adst, srcF, dst3)


def _elu(x):
    return jnp.where(x > 0.0, x, jnp.exp(x) - 1.0)


def _tc2_body(apl_ref, aph_ref, anl_ref, anh_ref, dp_ref, dn_ref,
              w1_ref, w2_ref, wd1_ref, bd1_ref, wd2_ref, bd2_ref,
              h2_ref, h2n_ref, rec_ref):
    w1l = w1_ref[0:HALF, :]
    w1h = w1_ref[HALF:IN_DIM, :]
    ivp = (1.0 / (dp_ref[...].reshape(-1) + jnp.float32(1e-16))).reshape(-1, 1)
    ivn = (1.0 / (dn_ref[...].reshape(-1) + jnp.float32(1e-16))).reshape(-1, 1)
    h1 = jnp.dot(apl_ref[0] * ivp, w1l, preferred_element_type=F32)
    h1 = h1 + jnp.dot(aph_ref[0] * ivp, w1h, preferred_element_type=F32)
    h2 = jnp.dot(_elu(h1), w2_ref[...], preferred_element_type=F32)
    h1n = jnp.dot(anl_ref[0] * ivn, w1l, preferred_element_type=F32)
    h1n = h1n + jnp.dot(anh_ref[0] * ivn, w1h, preferred_element_type=F32)
    h2n = jnp.dot(_elu(h1n), w2_ref[...], preferred_element_type=F32)
    r1 = _elu(jnp.dot(h2, wd1_ref[...], preferred_element_type=F32)
              + bd1_ref[...])
    rec = jnp.dot(r1, wd2_ref[...], preferred_element_type=F32) + bd2_ref[...]
    zpad = jnp.zeros((h2.shape[0], 128 - OUT), F32)
    h2_ref[...] = jnp.concatenate([h2, zpad], axis=1)
    h2n_ref[...] = jnp.concatenate([h2n, zpad], axis=1)
    rec_ref[...] = rec


def _tc2(aggP, denP, aggN, denN, W1, W2, Wd1, bd1r, Wd2, bd2r):
    blk = 1024
    lo = lambda i: (0, i, 0)
    hi = lambda i: (1, i, 0)
    dP = denP[0].reshape(NP // CW, CW)
    dN = denN[0].reshape(NP // CW, CW)

    def full(shape):
        return pl.BlockSpec(shape, lambda i: tuple(0 for _ in shape))

    return pl.pallas_call(
        _tc2_body,
        grid=(NP // blk,),
        in_specs=[
            pl.BlockSpec((1, blk, HALF), lo),
            pl.BlockSpec((1, blk, HALF), hi),
            pl.BlockSpec((1, blk, HALF), lo),
            pl.BlockSpec((1, blk, HALF), hi),
            pl.BlockSpec((blk // CW, CW), lambda i: (i, 0)),
            pl.BlockSpec((blk // CW, CW), lambda i: (i, 0)),
            full((IN_DIM, HID)),
            full((HID, OUT)),
            full((OUT, HID)),
            full((1, HID)),
            full((HID, IN_DIM)),
            full((1, IN_DIM)),
        ],
        out_specs=[
            pl.BlockSpec((blk, 128), lambda i: (i, 0)),
            pl.BlockSpec((blk, 128), lambda i: (i, 0)),
            pl.BlockSpec((blk, IN_DIM), lambda i: (i, 0)),
        ],
        out_shape=[
            jax.ShapeDtypeStruct((NP, 128), F32),
            jax.ShapeDtypeStruct((NP, 128), F32),
            jax.ShapeDtypeStruct((NP, IN_DIM), F32),
        ],
    )(aggP, aggP, aggN, aggN, dP, dN, W1, W2, Wd1, bd1r, Wd2, bd2r)


def _csl_sc_body(h2pad, src_h, dst_h, acc_out, cnt_out,
                 src2, dst_l, rbuf, ones_t, zden_t, didx2,
                 sem_a, sem_b, sem_c, sem_d, sem_e, acc_s, cnt_s):
    c = lax.axis_index("c")
    s = lax.axis_index("s")
    row0 = s * RPT
    w = c * NS + s
    EPW = NP // 2

    pltpu.sync_copy(src_h.at[w], src2)
    pltpu.sync_copy(dst_h.at[pl.ds(w * EPW, EPW)], dst_l)

    def zd(i, _):
        zden_t[pl.ds(i * 16, 16)] = _zvec()
        return 0
    lax.fori_loop(0, RPT // 16, zd, 0)
    pltpu.sync_copy(zden_t, cnt_s.at[pl.ds(row0, RPT)])

    def zr(i, _):
        for v in range(128 // 16):
            rbuf[i, pl.ds(v * 16, 16)] = _zvec()
        return 0
    lax.fori_loop(0, CW, zr, 0)
    for k in range(RPT // CW):
        pltpu.sync_copy(rbuf, acc_s.at[pl.ds(row0 + k * CW, CW)])

    for k in range(CW // 16):
        ones_t[pl.ds(k * 16, 16)] = jnp.ones((16,), F32)

    plsc.subcore_barrier()

    def firec(r, _):
        pltpu.async_copy(ones_t, cnt_s.at[src2.at[r]], sem_e, add=True)

        @pl.when(r >= 8)
        def _():
            pltpu.make_async_copy(ones_t, cnt_s.at[src2.at[0]], sem_e).wait()
        return 0
    lax.fori_loop(0, WPT, firec, 0)

    def _stage_didx(r, half):
        for k in range(4):
            didx2[half, pl.ds(k * 16, 16)] = src2[r, pl.ds(64 * half + k * 16, 16)]

    def _fire_g(g, half, sem):
        pltpu.async_copy(h2pad.at[dst_l.at[pl.ds(g * 64, 64)]],
                         rbuf.at[pl.ds(64 * half, 64)], sem)

    def _wait_g(sem):
        pltpu.make_async_copy(h2pad.at[pl.ds(0, 64)],
                              rbuf.at[pl.ds(0, 64)], sem).wait()

    def _fire_s(half, sem):
        pltpu.async_copy(rbuf.at[pl.ds(64 * half, 64)],
                         acc_s.at[didx2.at[half]], sem, add=True)

    def _wait_s(half, sem):
        pltpu.make_async_copy(rbuf.at[pl.ds(64 * half, 64)],
                              acc_s.at[didx2.at[half]], sem).wait()

    _stage_didx(0, 0)
    _fire_g(0, 0, sem_a)
    _stage_didx(0, 1)
    _fire_g(1, 1, sem_b)

    def p1(gg, _):
        _wait_g(sem_a)
        _fire_s(0, sem_c)
        _wait_g(sem_b)
        _fire_s(1, sem_d)

        @pl.when(gg < WPT - 1)
        def _():
            _wait_s(0, sem_c)
            _stage_didx(gg + 1, 0)
            _fire_g(2 * gg + 2, 0, sem_a)
            _wait_s(1, sem_d)
            _stage_didx(gg + 1, 1)
            _fire_g(2 * gg + 3, 1, sem_b)
        return 0
    lax.fori_loop(0, WPT, p1, 0)
    _wait_s(0, sem_c)
    _wait_s(1, sem_d)

    def drainc(r, _):
        pltpu.make_async_copy(ones_t, cnt_s.at[src2.at[0]], sem_e).wait()
        return 0
    lax.fori_loop(0, 8, drainc, 0)

    plsc.subcore_barrier()

    pltpu.sync_copy(acc_s.at[pl.ds(row0, RPT)], acc_out.at[c, pl.ds(row0, RPT)])
    pltpu.sync_copy(cnt_s.at[pl.ds(row0, RPT)], cnt_out.at[c, pl.ds(row0, RPT)])


def _csl_sc(h2pad, src3d, dstF):
    mesh = plsc.VectorSubcoreMesh(core_axis_name="c", subcore_axis_name="s")
    f = pl.kernel(
        _csl_sc_body,
        out_type=[
            jax.ShapeDtypeStruct((2, NP, 128), F32),
            jax.ShapeDtypeStruct((2, NP), F32),
        ],
        mesh=mesh,
        compiler_params=pltpu.CompilerParams(needs_layout_passes=False),
        scratch_types=dict(
            src2=pltpu.VMEM((WPT, CW), I32),
            dst_l=pltpu.VMEM((NP // 2,), I32),
            rbuf=pltpu.VMEM((CW, 128), F32),
            ones_t=pltpu.VMEM((CW,), F32),
            zden_t=pltpu.VMEM((RPT,), F32),
            didx2=pltpu.VMEM((2, 64), I32),
            sem_a=pltpu.SemaphoreType.DMA,
            sem_b=pltpu.SemaphoreType.DMA,
            sem_c=pltpu.SemaphoreType.DMA,
            sem_d=pltpu.SemaphoreType.DMA,
            sem_e=pltpu.SemaphoreType.DMA,
            acc_s=pltpu.VMEM_SHARED((NP, 128), F32),
            cnt_s=pltpu.VMEM_SHARED((NP,), F32),
        ),
    )
    return f(h2pad, src3d, dstF)


def _tc3_body(a_lo, a_hi, c_lo, c_hi, out_ref):
    cnt = (c_lo[0] + c_hi[0]).reshape(-1)
    inv = 1.0 / jnp.maximum(cnt, 1.0)
    out_ref[...] = (a_lo[0] + a_hi[0]) * inv.reshape(-1, 1)


def _tc3(acc2, cnt2):
    blk = 1024
    lo = lambda i: (0, i, 0)
    hi = lambda i: (1, i, 0)
    cnt3 = cnt2.reshape(2, NP // CW, CW)
    return pl.pallas_call(
        _tc3_body,
        grid=(NP // blk,),
        in_specs=[
            pl.BlockSpec((1, blk, 128), lo),
            pl.BlockSpec((1, blk, 128), hi),
            pl.BlockSpec((1, blk // CW, CW), lo),
            pl.BlockSpec((1, blk // CW, CW), hi),
        ],
        out_specs=pl.BlockSpec((blk, 128), lambda i: (i, 0)),
        out_shape=jax.ShapeDtypeStruct((NP, 128), F32),
    )(acc2, acc2, cnt3, cnt3)


def _pad_edges(idx):
    blocks = idx.reshape(NS, E // NS)
    blocks = jnp.pad(blocks, ((0, 0), (0, NP - E // NS)),
                     constant_values=NP - 1)
    return blocks.reshape(NS, CPT, CW)


def kernel(features, edge_index, edge_CSL, W1, att_src1, att_dst1, W2,
           Wd1, bd1, Wd2, bd2):
    att2p = jnp.zeros((HID, 128), F32)
    att2p = att2p.at[:, 0].set(att_src1).at[:, 1].set(att_dst1)
    a2 = _tc1(features, W1, att2p)
    asrc = jnp.pad(a2[:, 0], (0, NP - N))
    adst = jnp.pad(a2[:, 1], (0, NP - N))

    fcat = jnp.zeros((2 * NP, HALF), F32)
    fcat = fcat.at[0:N].set(features[:, :HALF])
    fcat = fcat.at[NP:NP + N].set(features[:, HALF:])

    srcP = _pad_edges(edge_index[0])
    dstP = _pad_edges(edge_index[1])
    srcN = _pad_edges(edge_CSL[0])
    dstN = _pad_edges(edge_CSL[1])

    aggP, denP = _gat_sc(fcat, asrc, adst, srcP.reshape(-1), dstP)
    aggN, denN = _gat_sc(fcat, asrc, adst, srcN.reshape(-1), dstN)

    h2p, h2np, recp = _tc2(aggP, denP, aggN, denN, W1, W2, Wd1,
                           bd1.reshape(1, HID), Wd2, bd2.reshape(1, IN_DIM))

    acc2, cnt2 = _csl_sc(h2p, srcP.reshape(2 * NS, WPT, CW),
                         dstP.reshape(-1))
    hp = _tc3(acc2, cnt2)

    return h2p[:N, :OUT], hp[:N, :OUT], h2np[:N, :OUT], recp[:N]


# speedup vs baseline: 10.0346x; 1.0284x over previous
import jax
import jax.numpy as jnp
from jax import lax
from jax.experimental import pallas as pl
from jax.experimental.pallas import tpu as pltpu
from jax.experimental.pallas import tpu_sc as plsc

N = 10000
E = 160000
IN_DIM, HID, OUT = 256, 512, 64
HALF = IN_DIM // 2
NS = 16
NP = 10240
RPT = NP // NS
CW = 128
CPT = NP // CW
WPT = NP // 2 // CW
F32 = jnp.float32
I32 = jnp.int32


def _zvec():
    return jnp.zeros((16,), F32)


def _tc1_body(x_ref, w1_ref, att_ref, out_ref):
    wmat = jnp.dot(w1_ref[...], att_ref[...], preferred_element_type=F32)
    out_ref[...] = jnp.dot(x_ref[...], wmat, preferred_element_type=F32)


def _tc1(features, W1, att2p):
    return pl.pallas_call(
        _tc1_body,
        grid=(25,),
        in_specs=[
            pl.BlockSpec((400, IN_DIM), lambda i: (i, 0)),
            pl.BlockSpec((IN_DIM, HID), lambda i: (0, 0)),
            pl.BlockSpec((HID, 128), lambda i: (0, 0)),
        ],
        out_specs=pl.BlockSpec((400, 128), lambda i: (i, 0)),
        out_shape=jax.ShapeDtypeStruct((N, 128), F32),
    )(features, W1, att2p)


def _gat_sc_body(fcat, asrc_h, adst_h, src_h, dst_h, out, den_out,
                 src_l, dst2, w_l, rbuf, didx2,
                 sem_a, sem_b, sem_c, sem_d, den_s, agg_s):
    c = lax.axis_index("c")
    s = lax.axis_index("s")
    row0 = s * RPT
    EP = NP

    pltpu.sync_copy(src_h.at[pl.ds(s * EP, EP)], src_l)
    pltpu.sync_copy(dst_h.at[s], dst2)

    def zd(i, _):
        w_l[pl.ds(i * 16, 16)] = _zvec()
        return 0
    lax.fori_loop(0, RPT // 16, zd, 0)
    pltpu.sync_copy(w_l.at[pl.ds(0, RPT)], den_s.at[pl.ds(row0, RPT)])

    def zr(i, _):
        for v in range(8):
            rbuf[i, pl.ds(v * 16, 16)] = _zvec()
        return 0
    lax.fori_loop(0, CW, zr, 0)
    for k in range(RPT // CW):
        pltpu.sync_copy(rbuf, agg_s.at[pl.ds(row0 + k * CW, CW)])

    plsc.subcore_barrier()

    def fire1(r, _):
        sl_e = pl.ds(r * CW, CW)
        pltpu.async_copy(asrc_h.at[src_l.at[sl_e]], w_l.at[sl_e], sem_a)
        pltpu.async_copy(adst_h.at[dst2.at[r]], rbuf.at[r], sem_b)

        @pl.when(r >= 8)
        def _():
            pltpu.make_async_copy(asrc_h.at[pl.ds(0, CW)],
                                  w_l.at[pl.ds(0, CW)], sem_a).wait()
            pltpu.make_async_copy(adst_h.at[pl.ds(0, CW)],
                                  rbuf.at[0], sem_b).wait()
        return 0
    lax.fori_loop(0, CPT, fire1, 0)

    def drain1(r, _):
        pltpu.make_async_copy(asrc_h.at[pl.ds(0, CW)],
                              w_l.at[pl.ds(0, CW)], sem_a).wait()
        pltpu.make_async_copy(adst_h.at[pl.ds(0, CW)], rbuf.at[0], sem_b).wait()
        return 0
    lax.fori_loop(0, 8, drain1, 0)

    def p1(i, _):
        sl = pl.ds(i * 16, 16)
        e = w_l[sl] + rbuf[i >> 3, pl.ds((i & 7) * 16, 16)]
        e = jnp.where(e > 0.0, e, e * jnp.float32(0.2))
        w_l[sl] = jnp.exp(e)
        return 0
    lax.fori_loop(0, EP // 16, p1, 0)

    def fired(r, _):
        pltpu.async_copy(w_l.at[pl.ds(r * CW, CW)], den_s.at[dst2.at[r]],
                         sem_a, add=True)

        @pl.when(r >= 8)
        def _():
            pltpu.make_async_copy(w_l.at[pl.ds(0, CW)],
                                  den_s.at[dst2.at[0]], sem_a).wait()
        return 0
    lax.fori_loop(0, CPT, fired, 0)

    def draind(r, _):
        pltpu.make_async_copy(w_l.at[pl.ds(0, CW)],
                              den_s.at[dst2.at[0]], sem_a).wait()
        return 0
    lax.fori_loop(0, 8, draind, 0)

    coff = c * NP

    def padj(i, _):
        sl = pl.ds(i * 16, 16)
        src_l[sl] = src_l[sl] + coff
        return 0
    lax.fori_loop(0, EP // 16, padj, 0)

    plsc.subcore_barrier()

    def _stage_didx(r, half):
        for k in range(4):
            didx2[half, pl.ds(k * 16, 16)] = dst2[r, pl.ds(64 * half + k * 16, 16)]

    def _fire_g(r, half, sem):
        pltpu.async_copy(fcat.at[src_l.at[pl.ds(r * CW + 64 * half, 64)]],
                         rbuf.at[pl.ds(64 * half, 64)], sem)

    def _wait_g(sem):
        pltpu.make_async_copy(fcat.at[pl.ds(0, 64)],
                              rbuf.at[pl.ds(0, 64)], sem).wait()

    def _fire_s(half, sem):
        pltpu.async_copy(rbuf.at[pl.ds(64 * half, 64)],
                         agg_s.at[didx2.at[half]], sem, add=True)

    def _wait_s(half, sem):
        pltpu.make_async_copy(rbuf.at[pl.ds(64 * half, 64)],
                              agg_s.at[didx2.at[half]], sem).wait()

    def _scale(r, half):
        for k in range(4):
            wv = w_l[pl.ds(r * CW + 64 * half + k * 16, 16)]
            for j in range(16):
                wj = wv[j]
                e = 64 * half + k * 16 + j
                for v in range(8):
                    sl = pl.ds(v * 16, 16)
                    rbuf[e, sl] = rbuf[e, sl] * wj

    _stage_didx(0, 0)
    _fire_g(0, 0, sem_a)
    _stage_didx(0, 1)
    _fire_g(0, 1, sem_b)

    def p3(gg, _):
        _wait_g(sem_a)
        _scale(gg, 0)
        _fire_s(0, sem_c)
        _wait_g(sem_b)
        _scale(gg, 1)
        _fire_s(1, sem_d)

        @pl.when(gg < CPT - 1)
        def _():
            _wait_s(0, sem_c)
            _stage_didx(gg + 1, 0)
            _fire_g(gg + 1, 0, sem_a)
            _wait_s(1, sem_d)
            _stage_didx(gg + 1, 1)
            _fire_g(gg + 1, 1, sem_b)
        return 0
    lax.fori_loop(0, CPT, p3, 0)
    _wait_s(0, sem_c)
    _wait_s(1, sem_d)

    plsc.subcore_barrier()

    pltpu.sync_copy(agg_s.at[pl.ds(row0, RPT)], out.at[c, pl.ds(row0, RPT)])
    pltpu.sync_copy(den_s.at[pl.ds(row0, RPT)], den_out.at[c, pl.ds(row0, RPT)])


def _gat_sc(fcat, asrc, adst, srcF, dst3):
    mesh = plsc.VectorSubcoreMesh(core_axis_name="c", subcore_axis_name="s")
    f = pl.kernel(
        _gat_sc_body,
        out_type=[
            jax.ShapeDtypeStruct((2, NP, HALF), F32),
            jax.ShapeDtypeStruct((2, NP), F32),
        ],
        mesh=mesh,
        compiler_params=pltpu.CompilerParams(needs_layout_passes=False),
        scratch_types=dict(
            src_l=pltpu.VMEM((NP,), I32),
            dst2=pltpu.VMEM((CPT, CW), I32),
            w_l=pltpu.VMEM((NP,), F32),
            rbuf=pltpu.VMEM((CW, HALF), F32),
            didx2=pltpu.VMEM((2, 64), I32),
            sem_a=pltpu.SemaphoreType.DMA,
            sem_b=pltpu.SemaphoreType.DMA,
            sem_c=pltpu.SemaphoreType.DMA,
            sem_d=pltpu.SemaphoreType.DMA,
            den_s=pltpu.VMEM_SHARED((NP,), F32),
            agg_s=pltpu.VMEM_SHARED((NP, HALF), F32),
        ),
    )
    return f(fcat, asrc, adst, srcF, dst3)


def _elu(x):
    return jnp.where(x > 0.0, x, jnp.exp(x) - 1.0)


def _tc2_body(al_ref, ah_ref, d_ref, w1_ref, w2_ref, h2_ref):
    w1l = w1_ref[0:HALF, :]
    w1h = w1_ref[HALF:IN_DIM, :]
    iv = (1.0 / (d_ref[...].reshape(-1) + jnp.float32(1e-16))).reshape(-1, 1)
    h1 = jnp.dot(al_ref[0] * iv, w1l, preferred_element_type=F32)
    h1 = h1 + jnp.dot(ah_ref[0] * iv, w1h, preferred_element_type=F32)
    h2 = jnp.dot(_elu(h1), w2_ref[...], preferred_element_type=F32)
    zpad = jnp.zeros((h2.shape[0], 128 - OUT), F32)
    h2_ref[...] = jnp.concatenate([h2, zpad], axis=1)


def _tc2(agg, den, W1, W2):
    blk = 1024
    lo = lambda i: (0, i, 0)
    hi = lambda i: (1, i, 0)
    d2 = den[0].reshape(NP // CW, CW)

    def full(shape):
        return pl.BlockSpec(shape, lambda i: tuple(0 for _ in shape))

    return pl.pallas_call(
        _tc2_body,
        grid=(NP // blk,),
        in_specs=[
            pl.BlockSpec((1, blk, HALF), lo),
            pl.BlockSpec((1, blk, HALF), hi),
            pl.BlockSpec((blk // CW, CW), lambda i: (i, 0)),
            full((IN_DIM, HID)),
            full((HID, OUT)),
        ],
        out_specs=pl.BlockSpec((blk, 128), lambda i: (i, 0)),
        out_shape=jax.ShapeDtypeStruct((NP, 128), F32),
    )(agg, agg, d2, W1, W2)


def _tcrec_body(h2_ref, wd1_ref, bd1_ref, wd2_ref, bd2_ref, rec_ref):
    h2 = h2_ref[:, 0:OUT]
    r1 = _elu(jnp.dot(h2, wd1_ref[...], preferred_element_type=F32)
              + bd1_ref[...])
    rec_ref[...] = (jnp.dot(r1, wd2_ref[...], preferred_element_type=F32)
                    + bd2_ref[...])


def _tcrec(h2p, Wd1, bd1r, Wd2, bd2r):
    blk = 1024

    def full(shape):
        return pl.BlockSpec(shape, lambda i: tuple(0 for _ in shape))

    return pl.pallas_call(
        _tcrec_body,
        grid=(NP // blk,),
        in_specs=[
            pl.BlockSpec((blk, 128), lambda i: (i, 0)),
            full((OUT, HID)),
            full((1, HID)),
            full((HID, IN_DIM)),
            full((1, IN_DIM)),
        ],
        out_specs=pl.BlockSpec((blk, IN_DIM), lambda i: (i, 0)),
        out_shape=jax.ShapeDtypeStruct((NP, IN_DIM), F32),
    )(h2p, Wd1, bd1r, Wd2, bd2r)


def _csl_sc_body(h2pad, src_h, dst_h, acc_out, cnt_out,
                 src2, dst_l, rbuf, ones_t, zden_t, didx2,
                 sem_a, sem_b, sem_c, sem_d, sem_e, acc_s, cnt_s):
    c = lax.axis_index("c")
    s = lax.axis_index("s")
    row0 = s * RPT
    w = c * NS + s
    EPW = NP // 2

    pltpu.sync_copy(src_h.at[w], src2)
    pltpu.sync_copy(dst_h.at[pl.ds(w * EPW, EPW)], dst_l)

    def zd(i, _):
        zden_t[pl.ds(i * 16, 16)] = _zvec()
        return 0
    lax.fori_loop(0, RPT // 16, zd, 0)
    pltpu.sync_copy(zden_t, cnt_s.at[pl.ds(row0, RPT)])

    def zr(i, _):
        for v in range(128 // 16):
            rbuf[i, pl.ds(v * 16, 16)] = _zvec()
        return 0
    lax.fori_loop(0, CW, zr, 0)
    for k in range(RPT // CW):
        pltpu.sync_copy(rbuf, acc_s.at[pl.ds(row0 + k * CW, CW)])

    for k in range(CW // 16):
        ones_t[pl.ds(k * 16, 16)] = jnp.ones((16,), F32)

    plsc.subcore_barrier()

    def firec(r, _):
        pltpu.async_copy(ones_t, cnt_s.at[src2.at[r]], sem_e, add=True)

        @pl.when(r >= 8)
        def _():
            pltpu.make_async_copy(ones_t, cnt_s.at[src2.at[0]], sem_e).wait()
        return 0
    lax.fori_loop(0, WPT, firec, 0)

    def _stage_didx(r, half):
        for k in range(4):
            didx2[half, pl.ds(k * 16, 16)] = src2[r, pl.ds(64 * half + k * 16, 16)]

    def _fire_g(g, half, sem):
        pltpu.async_copy(h2pad.at[dst_l.at[pl.ds(g * 64, 64)]],
                         rbuf.at[pl.ds(64 * half, 64)], sem)

    def _wait_g(sem):
        pltpu.make_async_copy(h2pad.at[pl.ds(0, 64)],
                              rbuf.at[pl.ds(0, 64)], sem).wait()

    def _fire_s(half, sem):
        pltpu.async_copy(rbuf.at[pl.ds(64 * half, 64)],
                         acc_s.at[didx2.at[half]], sem, add=True)

    def _wait_s(half, sem):
        pltpu.make_async_copy(rbuf.at[pl.ds(64 * half, 64)],
                              acc_s.at[didx2.at[half]], sem).wait()

    _stage_didx(0, 0)
    _fire_g(0, 0, sem_a)
    _stage_didx(0, 1)
    _fire_g(1, 1, sem_b)

    def p1(gg, _):
        _wait_g(sem_a)
        _fire_s(0, sem_c)
        _wait_g(sem_b)
        _fire_s(1, sem_d)

        @pl.when(gg < WPT - 1)
        def _():
            _wait_s(0, sem_c)
            _stage_didx(gg + 1, 0)
            _fire_g(2 * gg + 2, 0, sem_a)
            _wait_s(1, sem_d)
            _stage_didx(gg + 1, 1)
            _fire_g(2 * gg + 3, 1, sem_b)
        return 0
    lax.fori_loop(0, WPT, p1, 0)
    _wait_s(0, sem_c)
    _wait_s(1, sem_d)

    def drainc(r, _):
        pltpu.make_async_copy(ones_t, cnt_s.at[src2.at[0]], sem_e).wait()
        return 0
    lax.fori_loop(0, 8, drainc, 0)

    plsc.subcore_barrier()

    pltpu.sync_copy(acc_s.at[pl.ds(row0, RPT)], acc_out.at[c, pl.ds(row0, RPT)])
    pltpu.sync_copy(cnt_s.at[pl.ds(row0, RPT)], cnt_out.at[c, pl.ds(row0, RPT)])


def _csl_sc(h2pad, src3d, dstF):
    mesh = plsc.VectorSubcoreMesh(core_axis_name="c", subcore_axis_name="s")
    f = pl.kernel(
        _csl_sc_body,
        out_type=[
            jax.ShapeDtypeStruct((2, NP, 128), F32),
            jax.ShapeDtypeStruct((2, NP), F32),
        ],
        mesh=mesh,
        compiler_params=pltpu.CompilerParams(needs_layout_passes=False),
        scratch_types=dict(
            src2=pltpu.VMEM((WPT, CW), I32),
            dst_l=pltpu.VMEM((NP // 2,), I32),
            rbuf=pltpu.VMEM((CW, 128), F32),
            ones_t=pltpu.VMEM((CW,), F32),
            zden_t=pltpu.VMEM((RPT,), F32),
            didx2=pltpu.VMEM((2, 64), I32),
            sem_a=pltpu.SemaphoreType.DMA,
            sem_b=pltpu.SemaphoreType.DMA,
            sem_c=pltpu.SemaphoreType.DMA,
            sem_d=pltpu.SemaphoreType.DMA,
            sem_e=pltpu.SemaphoreType.DMA,
            acc_s=pltpu.VMEM_SHARED((NP, 128), F32),
            cnt_s=pltpu.VMEM_SHARED((NP,), F32),
        ),
    )
    return f(h2pad, src3d, dstF)


def _tc3_body(a_lo, a_hi, c_lo, c_hi, out_ref):
    cnt = (c_lo[0] + c_hi[0]).reshape(-1)
    inv = 1.0 / jnp.maximum(cnt, 1.0)
    out_ref[...] = (a_lo[0] + a_hi[0]) * inv.reshape(-1, 1)


def _tc3(acc2, cnt2):
    blk = 1024
    lo = lambda i: (0, i, 0)
    hi = lambda i: (1, i, 0)
    cnt3 = cnt2.reshape(2, NP // CW, CW)
    return pl.pallas_call(
        _tc3_body,
        grid=(NP // blk,),
        in_specs=[
            pl.BlockSpec((1, blk, 128), lo),
            pl.BlockSpec((1, blk, 128), hi),
            pl.BlockSpec((1, blk // CW, CW), lo),
            pl.BlockSpec((1, blk // CW, CW), hi),
        ],
        out_specs=pl.BlockSpec((blk, 128), lambda i: (i, 0)),
        out_shape=jax.ShapeDtypeStruct((NP, 128), F32),
    )(acc2, acc2, cnt3, cnt3)


def _pad_edges(idx):
    blocks = idx.reshape(NS, E // NS)
    blocks = jnp.pad(blocks, ((0, 0), (0, NP - E // NS)),
                     constant_values=NP - 1)
    return blocks.reshape(NS, CPT, CW)


def kernel(features, edge_index, edge_CSL, W1, att_src1, att_dst1, W2,
           Wd1, bd1, Wd2, bd2):
    att2p = jnp.zeros((HID, 128), F32)
    att2p = att2p.at[:, 0].set(att_src1).at[:, 1].set(att_dst1)
    a2 = _tc1(features, W1, att2p)
    asrc = jnp.pad(a2[:, 0], (0, NP - N))
    adst = jnp.pad(a2[:, 1], (0, NP - N))

    fcat = jnp.zeros((2 * NP, HALF), F32)
    fcat = fcat.at[0:N].set(features[:, :HALF])
    fcat = fcat.at[NP:NP + N].set(features[:, HALF:])

    srcP = _pad_edges(edge_index[0])
    dstP = _pad_edges(edge_index[1])
    srcN = _pad_edges(edge_CSL[0])
    dstN = _pad_edges(edge_CSL[1])

    aggP, denP = _gat_sc(fcat, asrc, adst, srcP.reshape(-1), dstP)
    h2p = _tc2(aggP, denP, W1, W2)
    recp = _tcrec(h2p, Wd1, bd1.reshape(1, HID), Wd2, bd2.reshape(1, IN_DIM))
    aggN, denN = _gat_sc(fcat, asrc, adst, srcN.reshape(-1), dstN)
    h2np = _tc2(aggN, denN, W1, W2)

    acc2, cnt2 = _csl_sc(h2p, srcP.reshape(2 * NS, WPT, CW),
                         dstP.reshape(-1))
    hp = _tc3(acc2, cnt2)

    return h2p[:N, :OUT], hp[:N, :OUT], h2np[:N, :OUT], recp[:N]
